# Initial kernel scaffold; baseline (speedup 1.0000x reference)
import jax
import jax.numpy as jnp
from jax.experimental import pallas as pl


def kernel(x, edge_index, W1, b1, W2, b2, W3, b3):
    raise NotImplementedError("write your pallas kernel here")


# speedup vs baseline: 12.2120x; 12.2120x over previous
import functools

import jax
import jax.numpy as jnp
from jax import lax
from jax.experimental import pallas as pl
from jax.experimental.pallas import tpu as pltpu
from jax.experimental.pallas import tpu_sc as plsc

N_NODES = 10000
N_EDGES = 320000
D_IN = 128
D_HID = 128
D_OUT = 64
DH = 64

NC = 2
NS = 16
NW = NC * NS
NP = 10240
RPT = NP // NS
K = 128
EP = 327680
NCHUNK = EP // (NW * K)

_MESH = plsc.VectorSubcoreMesh(core_axis_name="c", subcore_axis_name="s")
_SC_PARAMS = pltpu.CompilerParams(use_tc_tiling_on_sc=False)


def _wid(c, s):
  return s * NC + c


def _deg_body(dst_hbm, zero_hbm, out_hbm, idx_d, ones_v, degacc, isem):
  c = lax.axis_index("c")
  s = lax.axis_index("s")
  base = _wid(c, s) * NCHUNK
  cp = pltpu.async_copy(dst_hbm.at[pl.ds(base, NCHUNK)], idx_d, isem)
  for i in range(K // 16):
    ones_v[pl.ds(i * 16, 16)] = jnp.full((16,), 1.0, jnp.float32)
  pltpu.sync_copy(zero_hbm.at[pl.ds(s * RPT, RPT)], degacc.at[pl.ds(s * RPT, RPT)])
  cp.wait()
  plsc.subcore_barrier()

  @pl.loop(0, NCHUNK)
  def _(j):
    pltpu.sync_copy(ones_v, degacc.at[idx_d.at[j]], add=True)

  plsc.subcore_barrier()
  pltpu.sync_copy(degacc.at[pl.ds(s * RPT, RPT)], out_hbm.at[c].at[pl.ds(s * RPT, RPT)])


@functools.partial(
    pl.kernel,
    out_type=jax.ShapeDtypeStruct((NC, NP), jnp.float32),
    mesh=_MESH,
    scratch_types=[
        pltpu.VMEM((NCHUNK, K), jnp.int32),
        pltpu.VMEM((K,), jnp.float32),
        pltpu.VMEM_SHARED((NP,), jnp.float32),
        pltpu.SemaphoreType.DMA,
    ],
    compiler_params=_SC_PARAMS,
)
def _deg_kernel(dst_hbm, zero_hbm, out_hbm, idx_d, ones_v, degacc, isem):
  _deg_body(dst_hbm, zero_hbm, out_hbm, idx_d, ones_v, degacc, isem)


def _spmm_body(src_hbm, dst_hbm, pa_hbm, pb_hbm, zero_hbm, out_hbm,
               idx_s, idx_d, rows0, rows1, acc, isem0, isem1, gsem0, gsem1):
  c = lax.axis_index("c")
  s = lax.axis_index("s")
  base = _wid(c, s) * NCHUNK
  cps = pltpu.async_copy(src_hbm.at[pl.ds(base, NCHUNK)], idx_s, isem0)
  cpd = pltpu.async_copy(dst_hbm.at[pl.ds(base, NCHUNK)], idx_d, isem1)
  cps.wait()
  cpd.wait()

  rows = (rows0, rows1)
  gsem = (gsem0, gsem1)
  p_refs = (pa_hbm, pb_hbm)

  for h in range(2):
    p_hbm = p_refs[h]
    pltpu.sync_copy(zero_hbm.at[pl.ds(s * RPT, RPT)], acc.at[pl.ds(s * RPT, RPT)])
    plsc.subcore_barrier()

    pltpu.async_copy(p_hbm.at[idx_s.at[0]], rows0, gsem0)
    pltpu.async_copy(p_hbm.at[idx_s.at[1]], rows1, gsem1)

    @pl.loop(0, NCHUNK, step=2)
    def _(jj):
      for b in range(2):
        j = jj + b
        pltpu.make_async_copy(p_hbm.at[idx_s.at[0]], rows[b], gsem[b]).wait()
        pltpu.sync_copy(rows[b], acc.at[idx_d.at[j]], add=True)

        @pl.when(j + 2 < NCHUNK)
        def _():
          pltpu.async_copy(p_hbm.at[idx_s.at[j + 2]], rows[b], gsem[b])

    plsc.subcore_barrier()
    pltpu.sync_copy(acc.at[pl.ds(s * RPT, RPT)],
                    out_hbm.at[h].at[c].at[pl.ds(s * RPT, RPT)])
    plsc.subcore_barrier()


@functools.partial(
    pl.kernel,
    out_type=jax.ShapeDtypeStruct((2, NC, NP, DH), jnp.float32),
    mesh=_MESH,
    scratch_types=[
        pltpu.VMEM((NCHUNK, K), jnp.int32),
        pltpu.VMEM((NCHUNK, K), jnp.int32),
        pltpu.VMEM((K, DH), jnp.float32),
        pltpu.VMEM((K, DH), jnp.float32),
        pltpu.VMEM_SHARED((NP, DH), jnp.float32),
        pltpu.SemaphoreType.DMA,
        pltpu.SemaphoreType.DMA,
        pltpu.SemaphoreType.DMA,
        pltpu.SemaphoreType.DMA,
    ],
    compiler_params=_SC_PARAMS,
)
def _spmm_kernel(src_hbm, dst_hbm, pa_hbm, pb_hbm, zero_hbm, out_hbm,
                 idx_s, idx_d, rows0, rows1, acc, isem0, isem1, gsem0, gsem1):
  _spmm_body(src_hbm, dst_hbm, pa_hbm, pb_hbm, zero_hbm, out_hbm,
             idx_s, idx_d, rows0, rows1, acc, isem0, isem1, gsem0, gsem1)


_BR = 1280
_GRID = NP // _BR


def _rsqrt_deg(d_ref):
  deg = d_ref[:, 0:1] + d_ref[:, 1:2] + 1.0
  return lax.rsqrt(deg)


def _tc1_body(x_ref, w_ref, d_ref, oa_ref, ob_ref):
  q = _rsqrt_deg(d_ref)
  m = q * jnp.dot(x_ref[...], w_ref[...], preferred_element_type=jnp.float32)
  oa_ref[...] = m[:, :DH]
  ob_ref[...] = m[:, DH:]


def _tc2_body(sa0_ref, sa1_ref, sb0_ref, sb1_ref, pa_ref, pb_ref, d_ref,
              b_ref, w_ref, oa_ref, ob_ref):
  q = _rsqrt_deg(d_ref)
  ha = q * (sa0_ref[...] + sa1_ref[...] + pa_ref[...]) + b_ref[:, :DH]
  hb = q * (sb0_ref[...] + sb1_ref[...] + pb_ref[...]) + b_ref[:, DH:]
  h = jnp.maximum(jnp.concatenate([ha, hb], axis=1), 0.0)
  m = q * jnp.dot(h, w_ref[...], preferred_element_type=jnp.float32)
  oa_ref[...] = m[:, :DH]
  ob_ref[...] = m[:, DH:]


def _tc3_body(sa0_ref, sa1_ref, sb0_ref, sb1_ref, qa_ref, qb_ref, d_ref,
              b2_ref, b3_ref, mu_ref, lv_ref):
  q = _rsqrt_deg(d_ref)
  mu_ref[...] = q * (sa0_ref[...] + sa1_ref[...] + qa_ref[...]) + b2_ref[...]
  lv_ref[...] = q * (sb0_ref[...] + sb1_ref[...] + qb_ref[...]) + b3_ref[...]


def _row_spec(width):
  return pl.BlockSpec((_BR, width), lambda i: (i, 0))


def _full_spec(shape):
  return pl.BlockSpec(shape, lambda i: (0,) * len(shape))


_half_out = [jax.ShapeDtypeStruct((NP, DH), jnp.float32)] * 2


def _tc1(x_pad, W1, degT):
  return pl.pallas_call(
      _tc1_body,
      grid=(_GRID,),
      in_specs=[_row_spec(128), _full_spec((128, D_HID)), _row_spec(2)],
      out_specs=[_row_spec(DH), _row_spec(DH)],
      out_shape=_half_out,
  )(x_pad, W1, degT)


def _tc2(s1, pa, pb, degT, b1, Wc):
  return pl.pallas_call(
      _tc2_body,
      grid=(_GRID,),
      in_specs=[_row_spec(DH)] * 6 + [_row_spec(2),
                _full_spec((1, 128)), _full_spec((128, 128))],
      out_specs=[_row_spec(DH), _row_spec(DH)],
      out_shape=_half_out,
  )(s1[0, 0], s1[0, 1], s1[1, 0], s1[1, 1], pa, pb, degT, b1, Wc)


def _tc3(s2, qa, qb, degT, b2, b3):
  return pl.pallas_call(
      _tc3_body,
      grid=(_GRID,),
      in_specs=[_row_spec(DH)] * 6 + [_row_spec(2),
                _full_spec((1, D_OUT)), _full_spec((1, D_OUT))],
      out_specs=[_row_spec(D_OUT), _row_spec(D_OUT)],
      out_shape=_half_out,
  )(s2[0, 0], s2[0, 1], s2[1, 0], s2[1, 1], qa, qb, degT, b2, b3)


def kernel(x, edge_index, W1, b1, W2, b2, W3, b3):
  ei = edge_index.astype(jnp.int32)
  pad = jnp.full((EP - N_EDGES,), NP - 1, jnp.int32)
  src2d = jnp.concatenate([ei[0], pad]).reshape(EP // K, K)
  dst2d = jnp.concatenate([ei[1], pad]).reshape(EP // K, K)
  x_pad = jnp.pad(x, ((0, NP - N_NODES), (0, 0)))
  zero1 = jnp.zeros((NP,), jnp.float32)
  zero2 = jnp.zeros((NP, DH), jnp.float32)
  Wc = jnp.concatenate([W2, W3], axis=1)
  b1r = b1.reshape(1, D_HID)
  b2r = b2.reshape(1, D_OUT)
  b3r = b3.reshape(1, D_OUT)

  deg2 = _deg_kernel(dst2d, zero1)
  degT = deg2.T

  pa, pb = _tc1(x_pad, W1, degT)
  s1 = _spmm_kernel(src2d, dst2d, pa, pb, zero2)
  qa, qb = _tc2(s1, pa, pb, degT, b1r, Wc)
  s2 = _spmm_kernel(src2d, dst2d, qa, qb, zero2)
  mu, lv = _tc3(s2, qa, qb, degT, b2r, b3r)
  return (mu[:N_NODES], lv[:N_NODES])


# speedup vs baseline: 12.7777x; 1.0463x over previous
import functools

import jax
import jax.numpy as jnp
from jax import lax
from jax.experimental import pallas as pl
from jax.experimental.pallas import tpu as pltpu
from jax.experimental.pallas import tpu_sc as plsc

N_NODES = 10000
N_EDGES = 320000
D_IN = 128
D_HID = 128
D_OUT = 64
DH = 64

NC = 2
NS = 16
NW = NC * NS
NP = 10240
RPT = NP // NS
K = 128
EP = 327680
NCHUNK = EP // (NW * K)
NCH0 = 114
NCH1 = 2 * NCHUNK - NCH0
NCHT = 2560

_MESH = plsc.VectorSubcoreMesh(core_axis_name="c", subcore_axis_name="s")
_SC_PARAMS = pltpu.CompilerParams(use_tc_tiling_on_sc=False)


def _wid(c, s):
  return s * NC + c


def _deg_body(dst_hbm, zero_hbm, out_hbm, idx_d, ones_v, degacc, isem):
  c = lax.axis_index("c")
  s = lax.axis_index("s")
  base = _wid(c, s) * NCHUNK
  cp = pltpu.async_copy(dst_hbm.at[pl.ds(base, NCHUNK)], idx_d, isem)
  for i in range(K // 16):
    ones_v[pl.ds(i * 16, 16)] = jnp.full((16,), 1.0, jnp.float32)
  pltpu.sync_copy(zero_hbm.at[pl.ds(s * RPT, RPT)], degacc.at[pl.ds(s * RPT, RPT)])
  cp.wait()
  plsc.subcore_barrier()

  @pl.loop(0, NCHUNK)
  def _(j):
    pltpu.sync_copy(ones_v, degacc.at[idx_d.at[j]], add=True)

  plsc.subcore_barrier()
  pltpu.sync_copy(degacc.at[pl.ds(s * RPT, RPT)], out_hbm.at[c].at[pl.ds(s * RPT, RPT)])


@functools.partial(
    pl.kernel,
    out_type=jax.ShapeDtypeStruct((NC, NP), jnp.float32),
    mesh=_MESH,
    scratch_types=[
        pltpu.VMEM((NCHUNK, K), jnp.int32),
        pltpu.VMEM((K,), jnp.float32),
        pltpu.VMEM_SHARED((NP,), jnp.float32),
        pltpu.SemaphoreType.DMA,
    ],
    compiler_params=_SC_PARAMS,
)
def _deg_kernel(dst_hbm, zero_hbm, out_hbm, idx_d, ones_v, degacc, isem):
  _deg_body(dst_hbm, zero_hbm, out_hbm, idx_d, ones_v, degacc, isem)


def _spmm_body(src_hbm, dst_hbm, pa_hbm, pb_hbm, zero_hbm, out_hbm,
               idx_s, idx_d, rows0, rows1, acc, isem0, isem1, gsem0, gsem1):
  c = lax.axis_index("c")
  s = lax.axis_index("s")

  @pl.when(c == 0)
  def _():
    base = s * NCH0
    cps = pltpu.async_copy(src_hbm.at[pl.ds(base, NCH0)],
                           idx_s.at[pl.ds(0, NCH0)], isem0)
    cpd = pltpu.async_copy(dst_hbm.at[pl.ds(base, NCH0)],
                           idx_d.at[pl.ds(0, NCH0)], isem1)
    cps.wait()
    cpd.wait()

  @pl.when(c == 1)
  def _():
    base = NS * NCH0 + s * NCH1
    cps = pltpu.async_copy(src_hbm.at[pl.ds(base, NCH1)],
                           idx_s.at[pl.ds(0, NCH1)], isem0)
    cpd = pltpu.async_copy(dst_hbm.at[pl.ds(base, NCH1)],
                           idx_d.at[pl.ds(0, NCH1)], isem1)
    cps.wait()
    cpd.wait()

  nch = jnp.where(c == 0, NCH0, NCH1)
  rows = (rows0, rows1)
  gsem = (gsem0, gsem1)
  p_refs = (pa_hbm, pb_hbm)

  for h in range(2):
    p_hbm = p_refs[h]
    pltpu.sync_copy(zero_hbm.at[pl.ds(s * RPT, RPT)], acc.at[pl.ds(s * RPT, RPT)])
    plsc.subcore_barrier()

    pltpu.async_copy(p_hbm.at[idx_s.at[0]], rows0, gsem0)
    pltpu.async_copy(p_hbm.at[idx_s.at[1]], rows1, gsem1)

    @pl.loop(0, nch, step=2)
    def _(jj):
      for b in range(2):
        j = jj + b
        pltpu.make_async_copy(p_hbm.at[idx_s.at[0]], rows[b], gsem[b]).wait()
        pltpu.sync_copy(rows[b], acc.at[idx_d.at[j]], add=True)

        @pl.when(j + 2 < nch)
        def _():
          pltpu.async_copy(p_hbm.at[idx_s.at[j + 2]], rows[b], gsem[b])

    plsc.subcore_barrier()
    pltpu.sync_copy(acc.at[pl.ds(s * RPT, RPT)],
                    out_hbm.at[h].at[c].at[pl.ds(s * RPT, RPT)])
    plsc.subcore_barrier()


@functools.partial(
    pl.kernel,
    out_type=jax.ShapeDtypeStruct((2, NC, NP, DH), jnp.float32),
    mesh=_MESH,
    scratch_types=[
        pltpu.VMEM((NCH0, K), jnp.int32),
        pltpu.VMEM((NCH0, K), jnp.int32),
        pltpu.VMEM((K, DH), jnp.float32),
        pltpu.VMEM((K, DH), jnp.float32),
        pltpu.VMEM_SHARED((NP, DH), jnp.float32),
        pltpu.SemaphoreType.DMA,
        pltpu.SemaphoreType.DMA,
        pltpu.SemaphoreType.DMA,
        pltpu.SemaphoreType.DMA,
    ],
    compiler_params=_SC_PARAMS,
)
def _spmm_kernel(src_hbm, dst_hbm, pa_hbm, pb_hbm, zero_hbm, out_hbm,
                 idx_s, idx_d, rows0, rows1, acc, isem0, isem1, gsem0, gsem1):
  _spmm_body(src_hbm, dst_hbm, pa_hbm, pb_hbm, zero_hbm, out_hbm,
             idx_s, idx_d, rows0, rows1, acc, isem0, isem1, gsem0, gsem1)


_BR = 1280
_GRID = NP // _BR


def _rsqrt_deg(d_ref):
  deg = d_ref[:, 0:1] + d_ref[:, 1:2] + 1.0
  return lax.rsqrt(deg)


def _tc1_body(x_ref, w_ref, d_ref, oa_ref, ob_ref):
  q = _rsqrt_deg(d_ref)
  m = q * jnp.dot(x_ref[...], w_ref[...], preferred_element_type=jnp.float32)
  oa_ref[...] = m[:, :DH]
  ob_ref[...] = m[:, DH:]


def _tc2_body(sa0_ref, sa1_ref, sb0_ref, sb1_ref, pa_ref, pb_ref, d_ref,
              b_ref, w_ref, oa_ref, ob_ref):
  q = _rsqrt_deg(d_ref)
  ha = q * (sa0_ref[...] + sa1_ref[...] + pa_ref[...]) + b_ref[:, :DH]
  hb = q * (sb0_ref[...] + sb1_ref[...] + pb_ref[...]) + b_ref[:, DH:]
  h = jnp.maximum(jnp.concatenate([ha, hb], axis=1), 0.0)
  m = q * jnp.dot(h, w_ref[...], preferred_element_type=jnp.float32)
  oa_ref[...] = m[:, :DH]
  ob_ref[...] = m[:, DH:]


def _tc3_body(sa0_ref, sa1_ref, sb0_ref, sb1_ref, qa_ref, qb_ref, d_ref,
              b2_ref, b3_ref, mu_ref, lv_ref):
  q = _rsqrt_deg(d_ref)
  mu_ref[...] = q * (sa0_ref[...] + sa1_ref[...] + qa_ref[...]) + b2_ref[...]
  lv_ref[...] = q * (sb0_ref[...] + sb1_ref[...] + qb_ref[...]) + b3_ref[...]


def _row_spec(width):
  return pl.BlockSpec((_BR, width), lambda i: (i, 0))


def _full_spec(shape):
  return pl.BlockSpec(shape, lambda i: (0,) * len(shape))


_half_out = [jax.ShapeDtypeStruct((NP, DH), jnp.float32)] * 2


def _tc1(x_pad, W1, degT):
  return pl.pallas_call(
      _tc1_body,
      grid=(_GRID,),
      in_specs=[_row_spec(128), _full_spec((128, D_HID)), _row_spec(2)],
      out_specs=[_row_spec(DH), _row_spec(DH)],
      out_shape=_half_out,
  )(x_pad, W1, degT)


def _tc2(s1, pa, pb, degT, b1, Wc):
  return pl.pallas_call(
      _tc2_body,
      grid=(_GRID,),
      in_specs=[_row_spec(DH)] * 6 + [_row_spec(2),
                _full_spec((1, 128)), _full_spec((128, 128))],
      out_specs=[_row_spec(DH), _row_spec(DH)],
      out_shape=_half_out,
  )(s1[0, 0], s1[0, 1], s1[1, 0], s1[1, 1], pa, pb, degT, b1, Wc)


def _tc3(s2, qa, qb, degT, b2, b3):
  return pl.pallas_call(
      _tc3_body,
      grid=(_GRID,),
      in_specs=[_row_spec(DH)] * 6 + [_row_spec(2),
                _full_spec((1, D_OUT)), _full_spec((1, D_OUT))],
      out_specs=[_row_spec(D_OUT), _row_spec(D_OUT)],
      out_shape=_half_out,
  )(s2[0, 0], s2[0, 1], s2[1, 0], s2[1, 1], qa, qb, degT, b2, b3)


def kernel(x, edge_index, W1, b1, W2, b2, W3, b3):
  ei = edge_index.astype(jnp.int32)
  pad = jnp.full((EP - N_EDGES,), NP - 1, jnp.int32)
  src2d = jnp.concatenate([ei[0], pad]).reshape(EP // K, K)
  dst2d = jnp.concatenate([ei[1], pad]).reshape(EP // K, K)
  x_pad = jnp.pad(x, ((0, NP - N_NODES), (0, 0)))
  zero1 = jnp.zeros((NP,), jnp.float32)
  zero2 = jnp.zeros((NP, DH), jnp.float32)
  Wc = jnp.concatenate([W2, W3], axis=1)
  b1r = b1.reshape(1, D_HID)
  b2r = b2.reshape(1, D_OUT)
  b3r = b3.reshape(1, D_OUT)

  deg2 = _deg_kernel(dst2d, zero1)
  degT = deg2.T

  pa, pb = _tc1(x_pad, W1, degT)
  s1 = _spmm_kernel(src2d, dst2d, pa, pb, zero2)
  qa, qb = _tc2(s1, pa, pb, degT, b1r, Wc)
  s2 = _spmm_kernel(src2d, dst2d, qa, qb, zero2)
  mu, lv = _tc3(s2, qa, qb, degT, b2r, b3r)
  return (mu[:N_NODES], lv[:N_NODES])


# speedup vs baseline: 21.7797x; 1.7045x over previous
import functools

import jax
import jax.numpy as jnp
from jax import lax
from jax.experimental import pallas as pl
from jax.experimental.pallas import tpu as pltpu
from jax.experimental.pallas import tpu_sc as plsc

N_NODES = 10000
N_EDGES = 320000
D_IN = 128
D_HID = 128
D_OUT = 64
DQ = 32
NQ = 4

NC = 2
NS = 16
NW = NC * NS
NP = 10240
RPT = NP // NS
K = 128
EP = 327680
NCHUNK = EP // (NW * K)

_MESH = plsc.VectorSubcoreMesh(core_axis_name="c", subcore_axis_name="s")
_SC_PARAMS = pltpu.CompilerParams(use_tc_tiling_on_sc=False)


def _wid(c, s):
  return s * NC + c


def _deg_body(dst_hbm, zero_hbm, out_hbm, idx_d, ones_v, degacc, isem):
  c = lax.axis_index("c")
  s = lax.axis_index("s")
  base = _wid(c, s) * NCHUNK
  cp = pltpu.async_copy(dst_hbm.at[pl.ds(base, NCHUNK)], idx_d, isem)
  for i in range(K // 16):
    ones_v[pl.ds(i * 16, 16)] = jnp.full((16,), 1.0, jnp.float32)
  pltpu.sync_copy(zero_hbm.at[pl.ds(s * RPT, RPT)], degacc.at[pl.ds(s * RPT, RPT)])
  cp.wait()
  plsc.subcore_barrier()

  @pl.loop(0, NCHUNK)
  def _(j):
    pltpu.sync_copy(ones_v, degacc.at[idx_d.at[j]], add=True)

  plsc.subcore_barrier()
  pltpu.sync_copy(degacc.at[pl.ds(s * RPT, RPT)], out_hbm.at[c].at[pl.ds(s * RPT, RPT)])


@functools.partial(
    pl.kernel,
    out_type=jax.ShapeDtypeStruct((NC, NP), jnp.float32),
    mesh=_MESH,
    scratch_types=[
        pltpu.VMEM((NCHUNK, K), jnp.int32),
        pltpu.VMEM((K,), jnp.float32),
        pltpu.VMEM_SHARED((NP,), jnp.float32),
        pltpu.SemaphoreType.DMA,
    ],
    compiler_params=_SC_PARAMS,
)
def _deg_kernel(dst_hbm, zero_hbm, out_hbm, idx_d, ones_v, degacc, isem):
  _deg_body(dst_hbm, zero_hbm, out_hbm, idx_d, ones_v, degacc, isem)


def _spmm_body(src_hbm, dst_hbm, p4_hbm, out_hbm,
               idx_s, idx_d, rows0, rows1, zbuf, pq, acc,
               isem0, isem1, gsem0, gsem1):
  c = lax.axis_index("c")
  s = lax.axis_index("s")
  base = _wid(c, s) * NCHUNK
  cps = pltpu.async_copy(src_hbm.at[pl.ds(base, NCHUNK)], idx_s, isem0)
  cpd = pltpu.async_copy(dst_hbm.at[pl.ds(base, NCHUNK)], idx_d, isem1)

  @pl.loop(0, RPT)
  def _(r):
    for cc in range(DQ // 16):
      zbuf[r, pl.ds(cc * 16, 16)] = jnp.zeros((16,), jnp.float32)

  cps.wait()
  cpd.wait()

  rows = (rows0, rows1)
  gsem = (gsem0, gsem1)

  for q in range(NQ):
    pltpu.sync_copy(p4_hbm.at[q].at[pl.ds(s * RPT, RPT)], pq.at[pl.ds(s * RPT, RPT)])
    pltpu.sync_copy(zbuf, acc.at[pl.ds(s * RPT, RPT)])
    plsc.subcore_barrier()

    pltpu.async_copy(pq.at[idx_s.at[0]], rows0, gsem0)
    pltpu.async_copy(pq.at[idx_s.at[1]], rows1, gsem1)

    @pl.loop(0, NCHUNK, step=2)
    def _(jj):
      for b in range(2):
        j = jj + b
        pltpu.make_async_copy(pq.at[idx_s.at[0]], rows[b], gsem[b]).wait()
        pltpu.sync_copy(rows[b], acc.at[idx_d.at[j]], add=True)

        @pl.when(j + 2 < NCHUNK)
        def _():
          pltpu.async_copy(pq.at[idx_s.at[j + 2]], rows[b], gsem[b])

    plsc.subcore_barrier()
    pltpu.sync_copy(acc.at[pl.ds(s * RPT, RPT)],
                    out_hbm.at[q].at[c].at[pl.ds(s * RPT, RPT)])
    plsc.subcore_barrier()


@functools.partial(
    pl.kernel,
    out_type=jax.ShapeDtypeStruct((NQ, NC, NP, DQ), jnp.float32),
    mesh=_MESH,
    scratch_types=[
        pltpu.VMEM((NCHUNK, K), jnp.int32),
        pltpu.VMEM((NCHUNK, K), jnp.int32),
        pltpu.VMEM((K, DQ), jnp.float32),
        pltpu.VMEM((K, DQ), jnp.float32),
        pltpu.VMEM((RPT, DQ), jnp.float32),
        pltpu.VMEM_SHARED((NP, DQ), jnp.float32),
        pltpu.VMEM_SHARED((NP, DQ), jnp.float32),
        pltpu.SemaphoreType.DMA,
        pltpu.SemaphoreType.DMA,
        pltpu.SemaphoreType.DMA,
        pltpu.SemaphoreType.DMA,
    ],
    compiler_params=_SC_PARAMS,
)
def _spmm_kernel(src_hbm, dst_hbm, p4_hbm, out_hbm,
                 idx_s, idx_d, rows0, rows1, zbuf, pq, acc,
                 isem0, isem1, gsem0, gsem1):
  _spmm_body(src_hbm, dst_hbm, p4_hbm, out_hbm,
             idx_s, idx_d, rows0, rows1, zbuf, pq, acc,
             isem0, isem1, gsem0, gsem1)


_BR = 1280
_GRID = NP // _BR


def _rsqrt_deg(d_ref):
  deg = d_ref[:, 0:1] + d_ref[:, 1:2] + 1.0
  return lax.rsqrt(deg)


def _tc1_body(x_ref, w_ref, d_ref, o_ref):
  q = _rsqrt_deg(d_ref)
  m = q * jnp.dot(x_ref[...], w_ref[...], preferred_element_type=jnp.float32)
  for i in range(NQ):
    o_ref[i] = m[:, i * DQ:(i + 1) * DQ]


def _tc2_body(s_ref, p_ref, d_ref, b_ref, w_ref, o_ref):
  q = _rsqrt_deg(d_ref)
  parts = [s_ref[i, 0] + s_ref[i, 1] + p_ref[i] for i in range(NQ)]
  h = q * jnp.concatenate(parts, axis=1) + b_ref[...]
  h = jnp.maximum(h, 0.0)
  m = q * jnp.dot(h, w_ref[...], preferred_element_type=jnp.float32)
  for i in range(NQ):
    o_ref[i] = m[:, i * DQ:(i + 1) * DQ]


def _tc3_body(s_ref, p_ref, d_ref, b2_ref, b3_ref, mu_ref, lv_ref):
  q = _rsqrt_deg(d_ref)
  parts = [s_ref[i, 0] + s_ref[i, 1] + p_ref[i] for i in range(NQ)]
  t = q * jnp.concatenate(parts, axis=1)
  mu_ref[...] = t[:, :D_OUT] + b2_ref[...]
  lv_ref[...] = t[:, D_OUT:] + b3_ref[...]


def _row_spec(width):
  return pl.BlockSpec((_BR, width), lambda i: (i, 0))


def _full_spec(shape):
  return pl.BlockSpec(shape, lambda i: (0,) * len(shape))


def _q_spec():
  return pl.BlockSpec((NQ, _BR, DQ), lambda i: (0, i, 0))


def _s_spec():
  return pl.BlockSpec((NQ, NC, _BR, DQ), lambda i: (0, 0, i, 0))


_q_out = jax.ShapeDtypeStruct((NQ, NP, DQ), jnp.float32)


def _tc1(x_pad, W1, degT):
  return pl.pallas_call(
      _tc1_body,
      grid=(_GRID,),
      in_specs=[_row_spec(128), _full_spec((128, D_HID)), _row_spec(2)],
      out_specs=_q_spec(),
      out_shape=_q_out,
  )(x_pad, W1, degT)


def _tc2(s1, p4, degT, b1, Wc):
  return pl.pallas_call(
      _tc2_body,
      grid=(_GRID,),
      in_specs=[_s_spec(), _q_spec(), _row_spec(2),
                _full_spec((1, 128)), _full_spec((128, 128))],
      out_specs=_q_spec(),
      out_shape=_q_out,
  )(s1, p4, degT, b1, Wc)


def _tc3(s2, q4, degT, b2, b3):
  return pl.pallas_call(
      _tc3_body,
      grid=(_GRID,),
      in_specs=[_s_spec(), _q_spec(), _row_spec(2),
                _full_spec((1, D_OUT)), _full_spec((1, D_OUT))],
      out_specs=[_row_spec(D_OUT), _row_spec(D_OUT)],
      out_shape=[jax.ShapeDtypeStruct((NP, D_OUT), jnp.float32)] * 2,
  )(s2, q4, degT, b2, b3)


def kernel(x, edge_index, W1, b1, W2, b2, W3, b3):
  ei = edge_index.astype(jnp.int32)
  pad = jnp.full((EP - N_EDGES,), NP - 1, jnp.int32)
  src2d = jnp.concatenate([ei[0], pad]).reshape(EP // K, K)
  dst2d = jnp.concatenate([ei[1], pad]).reshape(EP // K, K)
  x_pad = jnp.pad(x, ((0, NP - N_NODES), (0, 0)))
  zero1 = jnp.zeros((NP,), jnp.float32)
  Wc = jnp.concatenate([W2, W3], axis=1)
  b1r = b1.reshape(1, D_HID)
  b2r = b2.reshape(1, D_OUT)
  b3r = b3.reshape(1, D_OUT)

  deg2 = _deg_kernel(dst2d, zero1)
  degT = deg2.T

  p4 = _tc1(x_pad, W1, degT)
  s1 = _spmm_kernel(src2d, dst2d, p4)
  q4 = _tc2(s1, p4, degT, b1r, Wc)
  s2 = _spmm_kernel(src2d, dst2d, q4)
  mu, lv = _tc3(s2, q4, degT, b2r, b3r)
  return (mu[:N_NODES], lv[:N_NODES])


# speedup vs baseline: 23.7255x; 1.0893x over previous
import functools

import jax
import jax.numpy as jnp
from jax import lax
from jax.experimental import pallas as pl
from jax.experimental.pallas import tpu as pltpu
from jax.experimental.pallas import tpu_sc as plsc

N_NODES = 10000
N_EDGES = 320000
D_IN = 128
D_HID = 128
D_OUT = 64
DQ = 32
NQ = 4

NC = 2
NS = 16
NW = NC * NS
NP = 10240
RPT = NP // NS
K = 128
NCHT = N_EDGES // K
NCH_LO = NCHT // NW
NREM = NCHT - NCH_LO * NW
NCH_HI = NCH_LO + 1
NCH_UP = NCH_LO + 2

_MESH = plsc.VectorSubcoreMesh(core_axis_name="c", subcore_axis_name="s")
_SC_PARAMS = pltpu.CompilerParams(use_tc_tiling_on_sc=False)


def _chunks(c, s):
  wid = s * NC + c
  base = wid * NCH_LO + jnp.minimum(wid, NREM)
  nch = jnp.where(wid < NREM, NCH_HI, NCH_LO)
  start = base * K
  start_dma = jnp.minimum(start, N_EDGES - NCH_UP * K)
  return start_dma, start - start_dma, nch


def _deg_body(edst_hbm, zero_hbm, out_hbm, idx_d, ones_v, degacc, isem):
  c = lax.axis_index("c")
  s = lax.axis_index("s")
  start_dma, off, nch = _chunks(c, s)
  cp = pltpu.async_copy(edst_hbm.at[pl.ds(start_dma, NCH_UP * K)], idx_d, isem)
  for i in range(K // 16):
    ones_v[pl.ds(i * 16, 16)] = jnp.full((16,), 1.0, jnp.float32)
  pltpu.sync_copy(zero_hbm.at[pl.ds(s * RPT, RPT)], degacc.at[pl.ds(s * RPT, RPT)])
  cp.wait()
  plsc.subcore_barrier()

  @pl.loop(0, NCH_UP)
  def _(j):
    @pl.when(j < nch)
    def _():
      pltpu.sync_copy(ones_v, degacc.at[idx_d.at[pl.ds(off + j * K, K)]], add=True)

  plsc.subcore_barrier()
  pltpu.sync_copy(degacc.at[pl.ds(s * RPT, RPT)], out_hbm.at[c].at[pl.ds(s * RPT, RPT)])


@functools.partial(
    pl.kernel,
    out_type=jax.ShapeDtypeStruct((NC, NP), jnp.float32),
    mesh=_MESH,
    scratch_types=[
        pltpu.VMEM((NCH_UP * K,), jnp.int32),
        pltpu.VMEM((K,), jnp.float32),
        pltpu.VMEM_SHARED((NP,), jnp.float32),
        pltpu.SemaphoreType.DMA,
    ],
    compiler_params=_SC_PARAMS,
)
def _deg_kernel(edst_hbm, zero_hbm, out_hbm, idx_d, ones_v, degacc, isem):
  _deg_body(edst_hbm, zero_hbm, out_hbm, idx_d, ones_v, degacc, isem)


def _spmm_body(esrc_hbm, edst_hbm, p4_hbm, out_hbm,
               idx_s, idx_d, rows0, rows1, zbuf, pq, acc,
               isem0, isem1, gsem0, gsem1):
  c = lax.axis_index("c")
  s = lax.axis_index("s")
  start_dma, off, nch = _chunks(c, s)
  cps = pltpu.async_copy(esrc_hbm.at[pl.ds(start_dma, NCH_UP * K)], idx_s, isem0)
  cpd = pltpu.async_copy(edst_hbm.at[pl.ds(start_dma, NCH_UP * K)], idx_d, isem1)

  @pl.loop(0, RPT)
  def _(r):
    for cc in range(DQ // 16):
      zbuf[r, pl.ds(cc * 16, 16)] = jnp.zeros((16,), jnp.float32)

  cps.wait()
  cpd.wait()

  rows = (rows0, rows1)
  gsem = (gsem0, gsem1)

  for q in range(NQ):
    pltpu.sync_copy(p4_hbm.at[q].at[pl.ds(s * RPT, RPT)], pq.at[pl.ds(s * RPT, RPT)])
    pltpu.sync_copy(zbuf, acc.at[pl.ds(s * RPT, RPT)])
    plsc.subcore_barrier()

    pltpu.async_copy(pq.at[idx_s.at[pl.ds(off, K)]], rows0, gsem0)
    pltpu.async_copy(pq.at[idx_s.at[pl.ds(off + K, K)]], rows1, gsem1)

    @pl.loop(0, NCH_UP, step=2)
    def _(jj):
      for b in range(2):
        j = jj + b

        @pl.when(j < nch)
        def _():
          pltpu.make_async_copy(pq.at[idx_s.at[pl.ds(0, K)]], rows[b], gsem[b]).wait()
          pltpu.sync_copy(rows[b], acc.at[idx_d.at[pl.ds(off + j * K, K)]], add=True)

        @pl.when(j + 2 < nch)
        def _():
          pltpu.async_copy(pq.at[idx_s.at[pl.ds(off + (j + 2) * K, K)]], rows[b], gsem[b])

    plsc.subcore_barrier()
    pltpu.sync_copy(acc.at[pl.ds(s * RPT, RPT)],
                    out_hbm.at[q].at[c].at[pl.ds(s * RPT, RPT)])
    plsc.subcore_barrier()


@functools.partial(
    pl.kernel,
    out_type=jax.ShapeDtypeStruct((NQ, NC, NP, DQ), jnp.float32),
    mesh=_MESH,
    scratch_types=[
        pltpu.VMEM((NCH_UP * K,), jnp.int32),
        pltpu.VMEM((NCH_UP * K,), jnp.int32),
        pltpu.VMEM((K, DQ), jnp.float32),
        pltpu.VMEM((K, DQ), jnp.float32),
        pltpu.VMEM((RPT, DQ), jnp.float32),
        pltpu.VMEM_SHARED((NP, DQ), jnp.float32),
        pltpu.VMEM_SHARED((NP, DQ), jnp.float32),
        pltpu.SemaphoreType.DMA,
        pltpu.SemaphoreType.DMA,
        pltpu.SemaphoreType.DMA,
        pltpu.SemaphoreType.DMA,
    ],
    compiler_params=_SC_PARAMS,
)
def _spmm_kernel(esrc_hbm, edst_hbm, p4_hbm, out_hbm,
                 idx_s, idx_d, rows0, rows1, zbuf, pq, acc,
                 isem0, isem1, gsem0, gsem1):
  _spmm_body(esrc_hbm, edst_hbm, p4_hbm, out_hbm,
             idx_s, idx_d, rows0, rows1, zbuf, pq, acc,
             isem0, isem1, gsem0, gsem1)


_BR = 1280
_GRID = NP // _BR
_BR3 = 1000
_GRID3 = N_NODES // _BR3


def _rsqrt_deg(d_ref):
  deg = d_ref[:, 0:1] + d_ref[:, 1:2] + 1.0
  return lax.rsqrt(deg)


def _tc1_body(x_ref, w_ref, d_ref, o_ref):
  q = _rsqrt_deg(d_ref)
  m = q * jnp.dot(x_ref[...], w_ref[...], preferred_element_type=jnp.float32)
  for i in range(NQ):
    o_ref[i] = m[:, i * DQ:(i + 1) * DQ]


def _tc2_body(s_ref, p_ref, d_ref, b_ref, w_ref, o_ref):
  q = _rsqrt_deg(d_ref)
  parts = [s_ref[i, 0] + s_ref[i, 1] + p_ref[i] for i in range(NQ)]
  h = q * jnp.concatenate(parts, axis=1) + b_ref[...]
  h = jnp.maximum(h, 0.0)
  m = q * jnp.dot(h, w_ref[...], preferred_element_type=jnp.float32)
  for i in range(NQ):
    o_ref[i] = m[:, i * DQ:(i + 1) * DQ]


def _tc3_body(s_ref, p_ref, d_ref, b2_ref, b3_ref, mu_ref, lv_ref):
  q = _rsqrt_deg(d_ref)
  parts = [s_ref[i, 0] + s_ref[i, 1] + p_ref[i] for i in range(NQ)]
  t = q * jnp.concatenate(parts, axis=1)
  mu_ref[...] = t[:, :D_OUT] + b2_ref[...]
  lv_ref[...] = t[:, D_OUT:] + b3_ref[...]


def _row_spec(width, br=_BR):
  return pl.BlockSpec((br, width), lambda i: (i, 0))


def _full_spec(shape):
  return pl.BlockSpec(shape, lambda i: (0,) * len(shape))


def _q_spec(br=_BR):
  return pl.BlockSpec((NQ, br, DQ), lambda i: (0, i, 0))


def _s_spec(br=_BR):
  return pl.BlockSpec((NQ, NC, br, DQ), lambda i: (0, 0, i, 0))


_q_out = jax.ShapeDtypeStruct((NQ, NP, DQ), jnp.float32)


def _tc1(x_pad, W1, degT):
  return pl.pallas_call(
      _tc1_body,
      grid=(_GRID,),
      in_specs=[_row_spec(128), _full_spec((128, D_HID)), _row_spec(2)],
      out_specs=_q_spec(),
      out_shape=_q_out,
  )(x_pad, W1, degT)


def _tc2(s1, p4, degT, b1, Wc):
  return pl.pallas_call(
      _tc2_body,
      grid=(_GRID,),
      in_specs=[_s_spec(), _q_spec(), _row_spec(2),
                _full_spec((1, 128)), _full_spec((128, 128))],
      out_specs=_q_spec(),
      out_shape=_q_out,
  )(s1, p4, degT, b1, Wc)


def _tc3(s2, q4, degT, b2, b3):
  return pl.pallas_call(
      _tc3_body,
      grid=(_GRID3,),
      in_specs=[_s_spec(_BR3), _q_spec(_BR3), _row_spec(2, _BR3),
                _full_spec((1, D_OUT)), _full_spec((1, D_OUT))],
      out_specs=[_row_spec(D_OUT, _BR3), _row_spec(D_OUT, _BR3)],
      out_shape=[jax.ShapeDtypeStruct((N_NODES, D_OUT), jnp.float32)] * 2,
  )(s2, q4, degT, b2, b3)


def kernel(x, edge_index, W1, b1, W2, b2, W3, b3):
  ei = edge_index.astype(jnp.int32)
  esrc = ei[0]
  edst = ei[1]
  x_pad = jnp.pad(x, ((0, NP - N_NODES), (0, 0)))
  zero1 = jnp.zeros((NP,), jnp.float32)
  b1r = b1.reshape(1, D_HID)
  b2r = b2.reshape(1, D_OUT)
  b3r = b3.reshape(1, D_OUT)
  Wc = jnp.concatenate([W2, W3], axis=1)

  deg2 = _deg_kernel(edst, zero1)
  degT = deg2.T

  p4 = _tc1(x_pad, W1, degT)
  s1 = _spmm_kernel(esrc, edst, p4)
  q4 = _tc2(s1, p4, degT, b1r, Wc)
  s2 = _spmm_kernel(esrc, edst, q4)
  return _tc3(s2, q4, degT, b2r, b3r)


# speedup vs baseline: 24.9039x; 1.0497x over previous
import functools

import jax
import jax.numpy as jnp
from jax import lax
from jax.experimental import pallas as pl
from jax.experimental.pallas import tpu as pltpu
from jax.experimental.pallas import tpu_sc as plsc

N_NODES = 10000
N_EDGES = 320000
D_IN = 128
D_HID = 128
D_OUT = 64
DQ = 32
NQ = 4

NC = 2
NS = 16
NW = NC * NS
NP = 10240
RPT = NP // NS
K = 128
NCHT = N_EDGES // K
NCH_LO = NCHT // NW
NREM = NCHT - NCH_LO * NW
NCH_HI = NCH_LO + 1
NCH_UP = NCH_LO + 2

_MESH = plsc.VectorSubcoreMesh(core_axis_name="c", subcore_axis_name="s")
_SC_PARAMS = pltpu.CompilerParams(use_tc_tiling_on_sc=False)


def _chunks(c, s):
  wid = s * NC + c
  base = wid * NCH_LO + jnp.minimum(wid, NREM)
  nch = jnp.where(wid < NREM, NCH_HI, NCH_LO)
  start = base * K
  start_dma = jnp.minimum(start, N_EDGES - NCH_UP * K)
  return start_dma, start - start_dma, nch


def _deg_body(edge_hbm, zero_hbm, out_hbm, idx_d, ones_v, degacc, isem):
  c = lax.axis_index("c")
  s = lax.axis_index("s")
  start_dma, off, nch = _chunks(c, s)
  cp = pltpu.async_copy(edge_hbm.at[1].at[pl.ds(start_dma, NCH_UP * K)], idx_d, isem)
  for i in range(K // 16):
    ones_v[pl.ds(i * 16, 16)] = jnp.full((16,), 1.0, jnp.float32)
  pltpu.sync_copy(zero_hbm.at[pl.ds(s * RPT, RPT)], degacc.at[pl.ds(s * RPT, RPT)])
  cp.wait()
  plsc.subcore_barrier()

  @pl.loop(0, NCH_UP)
  def _(j):
    @pl.when(j < nch)
    def _():
      pltpu.sync_copy(ones_v, degacc.at[idx_d.at[pl.ds(off + j * K, K)]], add=True)

  plsc.subcore_barrier()
  pltpu.sync_copy(degacc.at[pl.ds(s * RPT, RPT)], out_hbm.at[c].at[pl.ds(s * RPT, RPT)])


@functools.partial(
    pl.kernel,
    out_type=jax.ShapeDtypeStruct((NC, NP), jnp.float32),
    mesh=_MESH,
    scratch_types=[
        pltpu.VMEM((NCH_UP * K,), jnp.int32),
        pltpu.VMEM((K,), jnp.float32),
        pltpu.VMEM_SHARED((NP,), jnp.float32),
        pltpu.SemaphoreType.DMA,
    ],
    compiler_params=_SC_PARAMS,
)
def _deg_kernel(edge_hbm, zero_hbm, out_hbm, idx_d, ones_v, degacc, isem):
  _deg_body(edge_hbm, zero_hbm, out_hbm, idx_d, ones_v, degacc, isem)


def _spmm_body(edge_hbm, p4_hbm, out_hbm,
               idx_s, idx_d, rows0, rows1, zbuf, pq0, pq1, acc,
               isem0, isem1, gsem0, gsem1, ssem0, ssem1):
  c = lax.axis_index("c")
  s = lax.axis_index("s")
  start_dma, off, nch = _chunks(c, s)
  cps = pltpu.async_copy(edge_hbm.at[0].at[pl.ds(start_dma, NCH_UP * K)], idx_s, isem0)
  cpd = pltpu.async_copy(edge_hbm.at[1].at[pl.ds(start_dma, NCH_UP * K)], idx_d, isem1)

  @pl.loop(0, RPT)
  def _(r):
    for cc in range(DQ // 16):
      zbuf[r, pl.ds(cc * 16, 16)] = jnp.zeros((16,), jnp.float32)

  rows = (rows0, rows1)
  gsem = (gsem0, gsem1)
  pqs = (pq0, pq1)
  ssem = (ssem0, ssem1)
  rslice = pl.ds(s * RPT, RPT)

  def stage(q, sync):
    cp = pltpu.async_copy(p4_hbm.at[q].at[rslice], pqs[q % 2].at[rslice],
                          ssem[q % 2])
    if sync:
      cp.wait()

  stage(0, True)
  stage(1, False)
  pltpu.sync_copy(zbuf, acc.at[rslice])
  cps.wait()
  cpd.wait()
  plsc.subcore_barrier()

  for q in range(NQ):
    pq = pqs[q % 2]
    pltpu.async_copy(pq.at[idx_s.at[pl.ds(off, K)]], rows0, gsem0)
    pltpu.async_copy(pq.at[idx_s.at[pl.ds(off + K, K)]], rows1, gsem1)

    @pl.loop(0, NCH_UP, step=2)
    def _(jj):
      for b in range(2):
        j = jj + b

        @pl.when(j < nch)
        def _():
          pltpu.make_async_copy(pq.at[idx_s.at[pl.ds(0, K)]], rows[b], gsem[b]).wait()
          pltpu.sync_copy(rows[b], acc.at[idx_d.at[pl.ds(off + j * K, K)]], add=True)

        @pl.when(j + 2 < nch)
        def _():
          pltpu.async_copy(pq.at[idx_s.at[pl.ds(off + (j + 2) * K, K)]], rows[b], gsem[b])

    plsc.subcore_barrier()
    pltpu.sync_copy(acc.at[rslice], out_hbm.at[q].at[c].at[rslice])
    if q + 1 < NQ:
      pltpu.sync_copy(zbuf, acc.at[rslice])
      if q + 2 < NQ:
        stage(q + 2, False)
      pltpu.make_async_copy(p4_hbm.at[q + 1].at[rslice],
                            pqs[(q + 1) % 2].at[rslice], ssem[(q + 1) % 2]).wait()
      plsc.subcore_barrier()


@functools.partial(
    pl.kernel,
    out_type=jax.ShapeDtypeStruct((NQ, NC, NP, DQ), jnp.float32),
    mesh=_MESH,
    scratch_types=[
        pltpu.VMEM((NCH_UP * K,), jnp.int32),
        pltpu.VMEM((NCH_UP * K,), jnp.int32),
        pltpu.VMEM((K, DQ), jnp.float32),
        pltpu.VMEM((K, DQ), jnp.float32),
        pltpu.VMEM((RPT, DQ), jnp.float32),
        pltpu.VMEM_SHARED((NP, DQ), jnp.float32),
        pltpu.VMEM_SHARED((NP, DQ), jnp.float32),
        pltpu.VMEM_SHARED((NP, DQ), jnp.float32),
        pltpu.SemaphoreType.DMA,
        pltpu.SemaphoreType.DMA,
        pltpu.SemaphoreType.DMA,
        pltpu.SemaphoreType.DMA,
        pltpu.SemaphoreType.DMA,
        pltpu.SemaphoreType.DMA,
    ],
    compiler_params=_SC_PARAMS,
)
def _spmm_kernel(edge_hbm, p4_hbm, out_hbm,
                 idx_s, idx_d, rows0, rows1, zbuf, pq0, pq1, acc,
                 isem0, isem1, gsem0, gsem1, ssem0, ssem1):
  _spmm_body(edge_hbm, p4_hbm, out_hbm,
             idx_s, idx_d, rows0, rows1, zbuf, pq0, pq1, acc,
             isem0, isem1, gsem0, gsem1, ssem0, ssem1)


_BR = 1280
_GRID = NP // _BR
_BR3 = 1000
_GRID3 = N_NODES // _BR3


def _rsqrt_deg(d_ref):
  deg = d_ref[:, 0:1] + d_ref[:, 1:2] + 1.0
  return lax.rsqrt(deg)


def _tc1_body(x_ref, w_ref, d_ref, o_ref):
  q = _rsqrt_deg(d_ref)
  m = q * jnp.dot(x_ref[...], w_ref[...], preferred_element_type=jnp.float32)
  for i in range(NQ):
    o_ref[i] = m[:, i * DQ:(i + 1) * DQ]


def _tc2_body(s_ref, p_ref, d_ref, b_ref, w_ref, o_ref):
  q = _rsqrt_deg(d_ref)
  parts = [s_ref[i, 0] + s_ref[i, 1] + p_ref[i] for i in range(NQ)]
  h = q * jnp.concatenate(parts, axis=1) + b_ref[...]
  h = jnp.maximum(h, 0.0)
  m = q * jnp.dot(h, w_ref[...], preferred_element_type=jnp.float32)
  for i in range(NQ):
    o_ref[i] = m[:, i * DQ:(i + 1) * DQ]


def _tc3_body(s_ref, p_ref, d_ref, b2_ref, b3_ref, mu_ref, lv_ref):
  q = _rsqrt_deg(d_ref)
  parts = [s_ref[i, 0] + s_ref[i, 1] + p_ref[i] for i in range(NQ)]
  t = q * jnp.concatenate(parts, axis=1)
  mu_ref[...] = t[:, :D_OUT] + b2_ref[...]
  lv_ref[...] = t[:, D_OUT:] + b3_ref[...]


def _row_spec(width, br=_BR):
  return pl.BlockSpec((br, width), lambda i: (i, 0))


def _full_spec(shape):
  return pl.BlockSpec(shape, lambda i: (0,) * len(shape))


def _q_spec(br=_BR):
  return pl.BlockSpec((NQ, br, DQ), lambda i: (0, i, 0))


def _s_spec(br=_BR):
  return pl.BlockSpec((NQ, NC, br, DQ), lambda i: (0, 0, i, 0))


_q_out = jax.ShapeDtypeStruct((NQ, NP, DQ), jnp.float32)


def _tc1(x_pad, W1, degT):
  return pl.pallas_call(
      _tc1_body,
      grid=(_GRID,),
      in_specs=[_row_spec(128), _full_spec((128, D_HID)), _row_spec(2)],
      out_specs=_q_spec(),
      out_shape=_q_out,
  )(x_pad, W1, degT)


def _tc2(s1, p4, degT, b1, Wc):
  return pl.pallas_call(
      _tc2_body,
      grid=(_GRID,),
      in_specs=[_s_spec(), _q_spec(), _row_spec(2),
                _full_spec((1, 128)), _full_spec((128, 128))],
      out_specs=_q_spec(),
      out_shape=_q_out,
  )(s1, p4, degT, b1, Wc)


def _tc3(s2, q4, degT, b2, b3):
  return pl.pallas_call(
      _tc3_body,
      grid=(_GRID3,),
      in_specs=[_s_spec(_BR3), _q_spec(_BR3), _row_spec(2, _BR3),
                _full_spec((1, D_OUT)), _full_spec((1, D_OUT))],
      out_specs=[_row_spec(D_OUT, _BR3), _row_spec(D_OUT, _BR3)],
      out_shape=[jax.ShapeDtypeStruct((N_NODES, D_OUT), jnp.float32)] * 2,
  )(s2, q4, degT, b2, b3)


def kernel(x, edge_index, W1, b1, W2, b2, W3, b3):
  ei = edge_index.astype(jnp.int32)
  x_pad = jnp.pad(x, ((0, NP - N_NODES), (0, 0)))
  zero1 = jnp.zeros((NP,), jnp.float32)
  b1r = b1.reshape(1, D_HID)
  b2r = b2.reshape(1, D_OUT)
  b3r = b3.reshape(1, D_OUT)
  Wc = jnp.concatenate([W2, W3], axis=1)

  deg2 = _deg_kernel(ei, zero1)
  degT = deg2.T

  p4 = _tc1(x_pad, W1, degT)
  s1 = _spmm_kernel(ei, p4)
  q4 = _tc2(s1, p4, degT, b1r, Wc)
  s2 = _spmm_kernel(ei, q4)
  return _tc3(s2, q4, degT, b2r, b3r)


# speedup vs baseline: 27.2936x; 1.0960x over previous
import functools

import jax
import jax.numpy as jnp
from jax import lax
from jax.experimental import pallas as pl
from jax.experimental.pallas import tpu as pltpu
from jax.experimental.pallas import tpu_sc as plsc

N_NODES = 10000
N_EDGES = 320000
D_IN = 128
D_HID = 128
D_OUT = 64
DQ = 32
NQ = 4

NC = 2
NS = 16
NW = NC * NS
NP = 10240
RPT = NP // NS
K = 128
NCHT = N_EDGES // K
NCH_LO = NCHT // NW
NREM = NCHT - NCH_LO * NW
NCH_HI = NCH_LO + 1
NCH_UP = NCH_LO + 2

_MESH = plsc.VectorSubcoreMesh(core_axis_name="c", subcore_axis_name="s")
_SC_PARAMS = pltpu.CompilerParams(use_tc_tiling_on_sc=False)


def _chunks(c, s):
  wid = s * NC + c
  base = wid * NCH_LO + jnp.minimum(wid, NREM)
  nch = jnp.where(wid < NREM, NCH_HI, NCH_LO)
  start = base * K
  start_dma = jnp.minimum(start, N_EDGES - NCH_UP * K)
  return start_dma, start - start_dma, nch


def _deg_body(edge_hbm, zero_hbm, out_hbm, idx_d, ones_v, degacc, isem):
  c = lax.axis_index("c")
  s = lax.axis_index("s")
  start_dma, off, nch = _chunks(c, s)
  cp = pltpu.async_copy(edge_hbm.at[1].at[pl.ds(start_dma, NCH_UP * K)], idx_d, isem)
  for i in range(K // 16):
    ones_v[pl.ds(i * 16, 16)] = jnp.full((16,), 1.0, jnp.float32)
  pltpu.sync_copy(zero_hbm.at[pl.ds(s * RPT, RPT)], degacc.at[pl.ds(s * RPT, RPT)])
  cp.wait()
  plsc.subcore_barrier()

  @pl.loop(0, NCH_UP)
  def _(j):
    @pl.when(j < nch)
    def _():
      pltpu.sync_copy(ones_v, degacc.at[idx_d.at[pl.ds(off + j * K, K)]], add=True)

  plsc.subcore_barrier()
  pltpu.sync_copy(degacc.at[pl.ds(s * RPT, RPT)], out_hbm.at[c].at[pl.ds(s * RPT, RPT)])


@functools.partial(
    pl.kernel,
    out_type=jax.ShapeDtypeStruct((NC, NP), jnp.float32),
    mesh=_MESH,
    scratch_types=[
        pltpu.VMEM((NCH_UP * K,), jnp.int32),
        pltpu.VMEM((K,), jnp.float32),
        pltpu.VMEM_SHARED((NP,), jnp.float32),
        pltpu.SemaphoreType.DMA,
    ],
    compiler_params=_SC_PARAMS,
)
def _deg_kernel(edge_hbm, zero_hbm, out_hbm, idx_d, ones_v, degacc, isem):
  _deg_body(edge_hbm, zero_hbm, out_hbm, idx_d, ones_v, degacc, isem)


def _spmm_body(edge_hbm, p4_hbm, out_hbm,
               idx_s, idx_d, rows0, rows1, zbuf, pq0, pq1, acc,
               isem0, isem1, gsem0, gsem1, ssem0, ssem1):
  c = lax.axis_index("c")
  s = lax.axis_index("s")
  start_dma, off, nch = _chunks(c, s)
  cps = pltpu.async_copy(edge_hbm.at[0].at[pl.ds(start_dma, NCH_UP * K)], idx_s, isem0)
  cpd = pltpu.async_copy(edge_hbm.at[1].at[pl.ds(start_dma, NCH_UP * K)], idx_d, isem1)

  @pl.loop(0, RPT)
  def _(r):
    for cc in range(DQ // 16):
      zbuf[r, pl.ds(cc * 16, 16)] = jnp.zeros((16,), jnp.float32)

  rows = (rows0, rows1)
  gsem = (gsem0, gsem1)
  pqs = (pq0, pq1)
  ssem = (ssem0, ssem1)
  rslice = pl.ds(s * RPT, RPT)

  def stage(q, sync):
    cp = pltpu.async_copy(p4_hbm.at[rslice, pl.ds(q * DQ, DQ)],
                          pqs[q % 2].at[rslice], ssem[q % 2])
    if sync:
      cp.wait()

  stage(0, True)
  stage(1, False)
  pltpu.sync_copy(zbuf, acc.at[rslice])
  cps.wait()
  cpd.wait()
  plsc.subcore_barrier()

  for q in range(NQ):
    pq = pqs[q % 2]
    pltpu.async_copy(pq.at[idx_s.at[pl.ds(off, K)]], rows0, gsem0)
    pltpu.async_copy(pq.at[idx_s.at[pl.ds(off + K, K)]], rows1, gsem1)

    @pl.loop(0, NCH_UP, step=2)
    def _(jj):
      for b in range(2):
        j = jj + b

        @pl.when(j < nch)
        def _():
          pltpu.make_async_copy(pq.at[idx_s.at[pl.ds(0, K)]], rows[b], gsem[b]).wait()
          pltpu.sync_copy(rows[b], acc.at[idx_d.at[pl.ds(off + j * K, K)]], add=True)

        @pl.when(j + 2 < nch)
        def _():
          pltpu.async_copy(pq.at[idx_s.at[pl.ds(off + (j + 2) * K, K)]], rows[b], gsem[b])

    plsc.subcore_barrier()
    pltpu.sync_copy(acc.at[rslice], out_hbm.at[q].at[c].at[rslice])
    if q + 1 < NQ:
      pltpu.sync_copy(zbuf, acc.at[rslice])
      if q + 2 < NQ:
        stage(q + 2, False)
      pltpu.make_async_copy(p4_hbm.at[rslice, pl.ds((q + 1) * DQ, DQ)],
                            pqs[(q + 1) % 2].at[rslice], ssem[(q + 1) % 2]).wait()
      plsc.subcore_barrier()


@functools.partial(
    pl.kernel,
    out_type=jax.ShapeDtypeStruct((NQ, NC, NP, DQ), jnp.float32),
    mesh=_MESH,
    scratch_types=[
        pltpu.VMEM((NCH_UP * K,), jnp.int32),
        pltpu.VMEM((NCH_UP * K,), jnp.int32),
        pltpu.VMEM((K, DQ), jnp.float32),
        pltpu.VMEM((K, DQ), jnp.float32),
        pltpu.VMEM((RPT, DQ), jnp.float32),
        pltpu.VMEM_SHARED((NP, DQ), jnp.float32),
        pltpu.VMEM_SHARED((NP, DQ), jnp.float32),
        pltpu.VMEM_SHARED((NP, DQ), jnp.float32),
        pltpu.SemaphoreType.DMA,
        pltpu.SemaphoreType.DMA,
        pltpu.SemaphoreType.DMA,
        pltpu.SemaphoreType.DMA,
        pltpu.SemaphoreType.DMA,
        pltpu.SemaphoreType.DMA,
    ],
    compiler_params=_SC_PARAMS,
)
def _spmm_kernel(edge_hbm, p4_hbm, out_hbm,
                 idx_s, idx_d, rows0, rows1, zbuf, pq0, pq1, acc,
                 isem0, isem1, gsem0, gsem1, ssem0, ssem1):
  _spmm_body(edge_hbm, p4_hbm, out_hbm,
             idx_s, idx_d, rows0, rows1, zbuf, pq0, pq1, acc,
             isem0, isem1, gsem0, gsem1, ssem0, ssem1)


_BR = 1280
_GRID = NP // _BR
_BR3 = 1000
_GRID3 = N_NODES // _BR3


def _rsqrt_deg(d_ref):
  deg = d_ref[:, 0:1] + d_ref[:, 1:2] + 1.0
  return lax.rsqrt(deg)


def _tc1_body(x_ref, w_ref, d_ref, o_ref):
  q = _rsqrt_deg(d_ref)
  o_ref[...] = q * jnp.dot(x_ref[...], w_ref[...],
                           preferred_element_type=jnp.float32)


def _spmm_sum(s_ref, p_ref):
  parts = [s_ref[i, 0] + s_ref[i, 1] for i in range(NQ)]
  return jnp.concatenate(parts, axis=1) + p_ref[...]


def _tc2_body(s_ref, p_ref, d_ref, b_ref, w_ref, o_ref):
  q = _rsqrt_deg(d_ref)
  h = q * _spmm_sum(s_ref, p_ref) + b_ref[...]
  h = jnp.maximum(h, 0.0)
  o_ref[...] = q * jnp.dot(h, w_ref[...], preferred_element_type=jnp.float32)


def _tc3_body(s_ref, p_ref, d_ref, b2_ref, b3_ref, mu_ref, lv_ref):
  q = _rsqrt_deg(d_ref)
  t = q * _spmm_sum(s_ref, p_ref)
  mu_ref[...] = t[:, :D_OUT] + b2_ref[...]
  lv_ref[...] = t[:, D_OUT:] + b3_ref[...]


def _row_spec(width, br=_BR):
  return pl.BlockSpec((br, width), lambda i: (i, 0))


def _full_spec(shape):
  return pl.BlockSpec(shape, lambda i: (0,) * len(shape))


def _q_spec(br=_BR):
  return pl.BlockSpec((NQ, br, DQ), lambda i: (0, i, 0))


def _s_spec(br=_BR):
  return pl.BlockSpec((NQ, NC, br, DQ), lambda i: (0, 0, i, 0))


_p_out = jax.ShapeDtypeStruct((NP, D_HID), jnp.float32)


def _tc1(x_pad, W1, degT):
  return pl.pallas_call(
      _tc1_body,
      grid=(_GRID,),
      in_specs=[_row_spec(128), _full_spec((128, D_HID)), _row_spec(2)],
      out_specs=_row_spec(D_HID),
      out_shape=_p_out,
  )(x_pad, W1, degT)


def _tc2(s1, p, degT, b1, Wc):
  return pl.pallas_call(
      _tc2_body,
      grid=(_GRID,),
      in_specs=[_s_spec(), _row_spec(128), _row_spec(2),
                _full_spec((1, 128)), _full_spec((128, 128))],
      out_specs=_row_spec(D_HID),
      out_shape=_p_out,
  )(s1, p, degT, b1, Wc)


def _tc3(s2, q, degT, b2, b3):
  return pl.pallas_call(
      _tc3_body,
      grid=(_GRID3,),
      in_specs=[_s_spec(_BR3), _row_spec(128, _BR3), _row_spec(2, _BR3),
                _full_spec((1, D_OUT)), _full_spec((1, D_OUT))],
      out_specs=[_row_spec(D_OUT, _BR3), _row_spec(D_OUT, _BR3)],
      out_shape=[jax.ShapeDtypeStruct((N_NODES, D_OUT), jnp.float32)] * 2,
  )(s2, q, degT, b2, b3)


def kernel(x, edge_index, W1, b1, W2, b2, W3, b3):
  ei = edge_index.astype(jnp.int32)
  x_pad = jnp.pad(x, ((0, NP - N_NODES), (0, 0)))
  zero1 = jnp.zeros((NP,), jnp.float32)
  b1r = b1.reshape(1, D_HID)
  b2r = b2.reshape(1, D_OUT)
  b3r = b3.reshape(1, D_OUT)
  Wc = jnp.concatenate([W2, W3], axis=1)

  deg2 = _deg_kernel(ei, zero1)
  degT = deg2.T

  p4 = _tc1(x_pad, W1, degT)
  s1 = _spmm_kernel(ei, p4)
  q4 = _tc2(s1, p4, degT, b1r, Wc)
  s2 = _spmm_kernel(ei, q4)
  return _tc3(s2, q4, degT, b2r, b3r)


# speedup vs baseline: 31.4613x; 1.1527x over previous
import functools

import jax
import jax.numpy as jnp
from jax import lax
from jax.experimental import pallas as pl
from jax.experimental.pallas import tpu as pltpu
from jax.experimental.pallas import tpu_sc as plsc

N_NODES = 10000
N_EDGES = 320000
D_IN = 128
D_HID = 128
D_OUT = 64
DQ = 32
NQ = 4

NC = 2
NS = 16
NW = NC * NS
NP = 10240
RPT = NP // NS
K = 128
NCHT = N_EDGES // K
NCH_LO = NCHT // NW
NREM = NCHT - NCH_LO * NW
NCH_HI = NCH_LO + 1
NCH_UP = NCH_LO + 2

_MESH = plsc.VectorSubcoreMesh(core_axis_name="c", subcore_axis_name="s")
_SC_PARAMS = pltpu.CompilerParams(use_tc_tiling_on_sc=False)


def _chunks(c, s):
  wid = s * NC + c
  base = wid * NCH_LO + jnp.minimum(wid, NREM)
  nch = jnp.where(wid < NREM, NCH_HI, NCH_LO)
  start = base * K
  start_dma = jnp.minimum(start, N_EDGES - NCH_UP * K)
  return start_dma, start - start_dma, nch


def _deg_body(edge_hbm, zero_hbm, out_hbm, idx_d, ones_v, degacc, isem):
  c = lax.axis_index("c")
  s = lax.axis_index("s")
  start_dma, off, nch = _chunks(c, s)
  cp = pltpu.async_copy(edge_hbm.at[1].at[pl.ds(start_dma, NCH_UP * K)], idx_d, isem)
  for i in range(K // 16):
    ones_v[pl.ds(i * 16, 16)] = jnp.full((16,), 1.0, jnp.float32)
  pltpu.sync_copy(zero_hbm.at[pl.ds(s * RPT, RPT)], degacc.at[pl.ds(s * RPT, RPT)])
  cp.wait()
  plsc.subcore_barrier()

  @pl.loop(0, NCH_UP)
  def _(j):
    @pl.when(j < nch)
    def _():
      pltpu.sync_copy(ones_v, degacc.at[idx_d.at[pl.ds(off + j * K, K)]], add=True)

  plsc.subcore_barrier()
  pltpu.sync_copy(degacc.at[pl.ds(s * RPT, RPT)], out_hbm.at[c].at[pl.ds(s * RPT, RPT)])


@functools.partial(
    pl.kernel,
    out_type=jax.ShapeDtypeStruct((NC, NP), jnp.float32),
    mesh=_MESH,
    scratch_types=[
        pltpu.VMEM((NCH_UP * K,), jnp.int32),
        pltpu.VMEM((K,), jnp.float32),
        pltpu.VMEM_SHARED((NP,), jnp.float32),
        pltpu.SemaphoreType.DMA,
    ],
    compiler_params=_SC_PARAMS,
)
def _deg_kernel(edge_hbm, zero_hbm, out_hbm, idx_d, ones_v, degacc, isem):
  _deg_body(edge_hbm, zero_hbm, out_hbm, idx_d, ones_v, degacc, isem)


def _spmm_body(edge_hbm, p4_hbm, out_hbm,
               idx_s, idx_d, rows0, rows1, zbuf, pq0, pq1, acc,
               isem0, isem1, gsem0, gsem1, ssem0, ssem1):
  c = lax.axis_index("c")
  s = lax.axis_index("s")
  start_dma, off, nch = _chunks(c, s)
  cps = pltpu.async_copy(edge_hbm.at[0].at[pl.ds(start_dma, NCH_UP * K)], idx_s, isem0)
  cpd = pltpu.async_copy(edge_hbm.at[1].at[pl.ds(start_dma, NCH_UP * K)], idx_d, isem1)

  @pl.loop(0, RPT)
  def _(r):
    for cc in range(DQ // 16):
      zbuf[r, pl.ds(cc * 16, 16)] = jnp.zeros((16,), jnp.float32)

  rows = (rows0, rows1)
  gsem = (gsem0, gsem1)
  pqs = (pq0, pq1)
  ssem = (ssem0, ssem1)
  rslice = pl.ds(s * RPT, RPT)

  def stage(q, sync):
    cp = pltpu.async_copy(p4_hbm.at[rslice, pl.ds(q * DQ, DQ)],
                          pqs[q % 2].at[rslice], ssem[q % 2])
    if sync:
      cp.wait()

  stage(0, True)
  stage(1, False)
  pltpu.sync_copy(zbuf, acc.at[rslice])
  cps.wait()
  cpd.wait()
  plsc.subcore_barrier()

  for q in range(NQ):
    pq = pqs[q % 2]
    pltpu.async_copy(pq.at[idx_s.at[pl.ds(off, K)]], rows0, gsem0)
    pltpu.async_copy(pq.at[idx_s.at[pl.ds(off + K, K)]], rows1, gsem1)

    @pl.loop(0, NCH_UP, step=2)
    def _(jj):
      for b in range(2):
        j = jj + b

        @pl.when(j < nch)
        def _():
          pltpu.make_async_copy(pq.at[idx_s.at[pl.ds(0, K)]], rows[b], gsem[b]).wait()
          pltpu.sync_copy(rows[b], acc.at[idx_d.at[pl.ds(off + j * K, K)]], add=True)

        @pl.when(j + 2 < nch)
        def _():
          pltpu.async_copy(pq.at[idx_s.at[pl.ds(off + (j + 2) * K, K)]], rows[b], gsem[b])

    plsc.subcore_barrier()
    pltpu.sync_copy(acc.at[rslice], out_hbm.at[c].at[rslice, pl.ds(q * DQ, DQ)])
    if q + 1 < NQ:
      pltpu.sync_copy(zbuf, acc.at[rslice])
      if q + 2 < NQ:
        stage(q + 2, False)
      pltpu.make_async_copy(p4_hbm.at[rslice, pl.ds((q + 1) * DQ, DQ)],
                            pqs[(q + 1) % 2].at[rslice], ssem[(q + 1) % 2]).wait()
      plsc.subcore_barrier()


@functools.partial(
    pl.kernel,
    out_type=jax.ShapeDtypeStruct((NC, NP, NQ * DQ), jnp.float32),
    mesh=_MESH,
    scratch_types=[
        pltpu.VMEM((NCH_UP * K,), jnp.int32),
        pltpu.VMEM((NCH_UP * K,), jnp.int32),
        pltpu.VMEM((K, DQ), jnp.float32),
        pltpu.VMEM((K, DQ), jnp.float32),
        pltpu.VMEM((RPT, DQ), jnp.float32),
        pltpu.VMEM_SHARED((NP, DQ), jnp.float32),
        pltpu.VMEM_SHARED((NP, DQ), jnp.float32),
        pltpu.VMEM_SHARED((NP, DQ), jnp.float32),
        pltpu.SemaphoreType.DMA,
        pltpu.SemaphoreType.DMA,
        pltpu.SemaphoreType.DMA,
        pltpu.SemaphoreType.DMA,
        pltpu.SemaphoreType.DMA,
        pltpu.SemaphoreType.DMA,
    ],
    compiler_params=_SC_PARAMS,
)
def _spmm_kernel(edge_hbm, p4_hbm, out_hbm,
                 idx_s, idx_d, rows0, rows1, zbuf, pq0, pq1, acc,
                 isem0, isem1, gsem0, gsem1, ssem0, ssem1):
  _spmm_body(edge_hbm, p4_hbm, out_hbm,
             idx_s, idx_d, rows0, rows1, zbuf, pq0, pq1, acc,
             isem0, isem1, gsem0, gsem1, ssem0, ssem1)


_BR = 1280
_GRID = NP // _BR
_BR3 = 1000
_GRID3 = N_NODES // _BR3


def _rsqrt_deg(d_ref):
  deg = d_ref[:, 0:1] + d_ref[:, 1:2] + 1.0
  return lax.rsqrt(deg)


def _tc1_body(x_ref, w_ref, d_ref, o_ref):
  q = _rsqrt_deg(d_ref)
  o_ref[...] = q * jnp.dot(x_ref[...], w_ref[...],
                           preferred_element_type=jnp.float32)


def _spmm_sum(s_ref, p_ref):
  return s_ref[0] + s_ref[1] + p_ref[...]


def _tc2_body(s_ref, p_ref, d_ref, b_ref, w_ref, o_ref):
  q = _rsqrt_deg(d_ref)
  h = q * _spmm_sum(s_ref, p_ref) + b_ref[...]
  h = jnp.maximum(h, 0.0)
  o_ref[...] = q * jnp.dot(h, w_ref[...], preferred_element_type=jnp.float32)


def _tc3_body(s_ref, p_ref, d_ref, b2_ref, b3_ref, mu_ref, lv_ref):
  q = _rsqrt_deg(d_ref)
  t = q * _spmm_sum(s_ref, p_ref)
  mu_ref[...] = t[:, :D_OUT] + b2_ref[...]
  lv_ref[...] = t[:, D_OUT:] + b3_ref[...]


def _row_spec(width, br=_BR):
  return pl.BlockSpec((br, width), lambda i: (i, 0))


def _full_spec(shape):
  return pl.BlockSpec(shape, lambda i: (0,) * len(shape))


def _s_spec(br=_BR):
  return pl.BlockSpec((NC, br, NQ * DQ), lambda i: (0, i, 0))


_p_out = jax.ShapeDtypeStruct((NP, D_HID), jnp.float32)


def _tc1(x_pad, W1, degT):
  return pl.pallas_call(
      _tc1_body,
      grid=(_GRID,),
      in_specs=[_row_spec(128), _full_spec((128, D_HID)), _row_spec(2)],
      out_specs=_row_spec(D_HID),
      out_shape=_p_out,
  )(x_pad, W1, degT)


def _tc2(s1, p, degT, b1, Wc):
  return pl.pallas_call(
      _tc2_body,
      grid=(_GRID,),
      in_specs=[_s_spec(), _row_spec(128), _row_spec(2),
                _full_spec((1, 128)), _full_spec((128, 128))],
      out_specs=_row_spec(D_HID),
      out_shape=_p_out,
  )(s1, p, degT, b1, Wc)


def _tc3(s2, q, degT, b2, b3):
  return pl.pallas_call(
      _tc3_body,
      grid=(_GRID3,),
      in_specs=[_s_spec(_BR3), _row_spec(128, _BR3), _row_spec(2, _BR3),
                _full_spec((1, D_OUT)), _full_spec((1, D_OUT))],
      out_specs=[_row_spec(D_OUT, _BR3), _row_spec(D_OUT, _BR3)],
      out_shape=[jax.ShapeDtypeStruct((N_NODES, D_OUT), jnp.float32)] * 2,
  )(s2, q, degT, b2, b3)


def kernel(x, edge_index, W1, b1, W2, b2, W3, b3):
  ei = edge_index.astype(jnp.int32)
  x_pad = jnp.pad(x, ((0, NP - N_NODES), (0, 0)))
  zero1 = jnp.zeros((NP,), jnp.float32)
  b1r = b1.reshape(1, D_HID)
  b2r = b2.reshape(1, D_OUT)
  b3r = b3.reshape(1, D_OUT)
  Wc = jnp.concatenate([W2, W3], axis=1)

  deg2 = _deg_kernel(ei, zero1)
  degT = deg2.T

  p4 = _tc1(x_pad, W1, degT)
  s1 = _spmm_kernel(ei, p4)
  q4 = _tc2(s1, p4, degT, b1r, Wc)
  s2 = _spmm_kernel(ei, q4)
  return _tc3(s2, q4, degT, b2r, b3r)


# speedup vs baseline: 35.0930x; 1.1154x over previous
import functools

import jax
import jax.numpy as jnp
from jax import lax
from jax.experimental import pallas as pl
from jax.experimental.pallas import tpu as pltpu
from jax.experimental.pallas import tpu_sc as plsc

N_NODES = 10000
N_EDGES = 320000
D_IN = 128
D_HID = 128
D_OUT = 64
DQ = 32
NQ = 4

NC = 2
NS = 16
NW = NC * NS
NP = 10240
RPT = NP // NS
K = 128
NCHT = N_EDGES // K
NCH_LO = NCHT // NW
NREM = NCHT - NCH_LO * NW
NCH_HI = NCH_LO + 1
NCH_UP = NCH_LO + 2

_MESH = plsc.VectorSubcoreMesh(core_axis_name="c", subcore_axis_name="s")
_SC_PARAMS = pltpu.CompilerParams(use_tc_tiling_on_sc=False)


def _chunks(c, s):
  wid = s * NC + c
  base = wid * NCH_LO + jnp.minimum(wid, NREM)
  nch = jnp.where(wid < NREM, NCH_HI, NCH_LO)
  start = base * K
  start_dma = jnp.minimum(start, N_EDGES - NCH_UP * K)
  return start_dma, start - start_dma, nch


def _deg_body(edge_hbm, zero_hbm, out_hbm, idx_d, ones_v, degacc, isem):
  c = lax.axis_index("c")
  s = lax.axis_index("s")
  start_dma, off, nch = _chunks(c, s)
  cp = pltpu.async_copy(edge_hbm.at[1].at[pl.ds(start_dma, NCH_UP * K)], idx_d, isem)
  for i in range(K // 16):
    ones_v[pl.ds(i * 16, 16)] = jnp.full((16,), 1.0, jnp.float32)
  pltpu.sync_copy(zero_hbm.at[pl.ds(s * RPT, RPT)], degacc.at[pl.ds(s * RPT, RPT)])
  cp.wait()
  plsc.subcore_barrier()

  @pl.loop(0, NCH_UP)
  def _(j):
    @pl.when(j < nch)
    def _():
      pltpu.sync_copy(ones_v, degacc.at[idx_d.at[pl.ds(off + j * K, K)]], add=True)

  plsc.subcore_barrier()
  pltpu.sync_copy(degacc.at[pl.ds(s * RPT, RPT)], out_hbm.at[c].at[pl.ds(s * RPT, RPT)])


@functools.partial(
    pl.kernel,
    out_type=jax.ShapeDtypeStruct((NC, NP), jnp.float32),
    mesh=_MESH,
    scratch_types=[
        pltpu.VMEM((NCH_UP * K,), jnp.int32),
        pltpu.VMEM((K,), jnp.float32),
        pltpu.VMEM_SHARED((NP,), jnp.float32),
        pltpu.SemaphoreType.DMA,
    ],
    compiler_params=_SC_PARAMS,
)
def _deg_kernel(edge_hbm, zero_hbm, out_hbm, idx_d, ones_v, degacc, isem):
  _deg_body(edge_hbm, zero_hbm, out_hbm, idx_d, ones_v, degacc, isem)


def _spmm_body(edge_hbm, p4_hbm, out_hbm,
               idx_s, idx_d, rows0, rows1, rows2, rows3, zbuf, pq0, pq1, acc,
               isem0, isem1, gsem0, gsem1, gsem2, gsem3,
               csem0, csem1, csem2, csem3, ssem0, ssem1):
  c = lax.axis_index("c")
  s = lax.axis_index("s")
  start_dma, off, nch = _chunks(c, s)
  cps = pltpu.async_copy(edge_hbm.at[0].at[pl.ds(start_dma, NCH_UP * K)], idx_s, isem0)
  cpd = pltpu.async_copy(edge_hbm.at[1].at[pl.ds(start_dma, NCH_UP * K)], idx_d, isem1)

  @pl.loop(0, RPT)
  def _(r):
    for cc in range(DQ // 16):
      zbuf[r, pl.ds(cc * 16, 16)] = jnp.zeros((16,), jnp.float32)

  rows = (rows0, rows1, rows2, rows3)
  gsem = (gsem0, gsem1, gsem2, gsem3)
  csem = (csem0, csem1, csem2, csem3)
  pqs = (pq0, pq1)
  ssem = (ssem0, ssem1)
  rslice = pl.ds(s * RPT, RPT)

  def stage(q, sync):
    cp = pltpu.async_copy(p4_hbm.at[rslice, pl.ds(q * DQ, DQ)],
                          pqs[q % 2].at[rslice], ssem[q % 2])
    if sync:
      cp.wait()

  stage(0, True)
  stage(1, False)
  pltpu.sync_copy(zbuf, acc.at[rslice])
  cps.wait()
  cpd.wait()
  plsc.subcore_barrier()

  def wait_gather(b):
    pltpu.make_async_copy(pqs[0].at[idx_s.at[pl.ds(0, K)]], rows[b],
                          gsem[b]).wait()

  def wait_scatter(b):
    pltpu.make_async_copy(rows[b], acc.at[idx_d.at[pl.ds(0, K)]],
                          csem[b]).wait()

  for q in range(NQ):
    pq = pqs[q % 2]
    pltpu.async_copy(pq.at[idx_s.at[pl.ds(off, K)]], rows0, gsem0)
    pltpu.async_copy(pq.at[idx_s.at[pl.ds(off + K, K)]], rows1, gsem1)

    @pl.loop(0, NCH_UP, step=4)
    def _(jj):
      for b in range(4):
        j = jj + b

        @pl.when(j < nch)
        def _():
          wait_gather(b)
          pltpu.async_copy(rows[b], acc.at[idx_d.at[pl.ds(off + j * K, K)]],
                           csem[b], add=True)

        @pl.when(j + 2 < nch)
        def _():
          @pl.when(j >= 2)
          def _():
            wait_scatter((b + 2) % 4)

          pltpu.async_copy(pq.at[idx_s.at[pl.ds(off + (j + 2) * K, K)]],
                           rows[(b + 2) % 4], gsem[(b + 2) % 4])

    for b in range(4):
      wait_scatter(b)

    plsc.subcore_barrier()
    pltpu.sync_copy(acc.at[rslice], out_hbm.at[c].at[rslice, pl.ds(q * DQ, DQ)])
    if q + 1 < NQ:
      pltpu.sync_copy(zbuf, acc.at[rslice])
      if q + 2 < NQ:
        stage(q + 2, False)
      pltpu.make_async_copy(p4_hbm.at[rslice, pl.ds((q + 1) * DQ, DQ)],
                            pqs[(q + 1) % 2].at[rslice], ssem[(q + 1) % 2]).wait()
      plsc.subcore_barrier()


@functools.partial(
    pl.kernel,
    out_type=jax.ShapeDtypeStruct((NC, NP, NQ * DQ), jnp.float32),
    mesh=_MESH,
    scratch_types=[
        pltpu.VMEM((NCH_UP * K,), jnp.int32),
        pltpu.VMEM((NCH_UP * K,), jnp.int32),
        pltpu.VMEM((K, DQ), jnp.float32),
        pltpu.VMEM((K, DQ), jnp.float32),
        pltpu.VMEM((K, DQ), jnp.float32),
        pltpu.VMEM((K, DQ), jnp.float32),
        pltpu.VMEM((RPT, DQ), jnp.float32),
        pltpu.VMEM_SHARED((NP, DQ), jnp.float32),
        pltpu.VMEM_SHARED((NP, DQ), jnp.float32),
        pltpu.VMEM_SHARED((NP, DQ), jnp.float32),
    ] + [pltpu.SemaphoreType.DMA] * 12,
    compiler_params=_SC_PARAMS,
)
def _spmm_kernel(edge_hbm, p4_hbm, out_hbm,
                 idx_s, idx_d, rows0, rows1, rows2, rows3, zbuf, pq0, pq1, acc,
                 isem0, isem1, gsem0, gsem1, gsem2, gsem3,
                 csem0, csem1, csem2, csem3, ssem0, ssem1):
  _spmm_body(edge_hbm, p4_hbm, out_hbm,
             idx_s, idx_d, rows0, rows1, rows2, rows3, zbuf, pq0, pq1, acc,
             isem0, isem1, gsem0, gsem1, gsem2, gsem3,
             csem0, csem1, csem2, csem3, ssem0, ssem1)


_BR = 1280
_GRID = NP // _BR
_BR3 = 1000
_GRID3 = N_NODES // _BR3


def _rsqrt_deg(d_ref):
  deg = d_ref[:, 0:1] + d_ref[:, 1:2] + 1.0
  return lax.rsqrt(deg)


def _tc1_body(x_ref, w_ref, d_ref, o_ref):
  q = _rsqrt_deg(d_ref)
  o_ref[...] = q * jnp.dot(x_ref[...], w_ref[...],
                           preferred_element_type=jnp.float32)


def _spmm_sum(s_ref, p_ref):
  return s_ref[0] + s_ref[1] + p_ref[...]


def _tc2_body(s_ref, p_ref, d_ref, b_ref, w_ref, o_ref):
  q = _rsqrt_deg(d_ref)
  h = q * _spmm_sum(s_ref, p_ref) + b_ref[...]
  h = jnp.maximum(h, 0.0)
  o_ref[...] = q * jnp.dot(h, w_ref[...], preferred_element_type=jnp.float32)


def _tc3_body(s_ref, p_ref, d_ref, b2_ref, b3_ref, mu_ref, lv_ref):
  q = _rsqrt_deg(d_ref)
  t = q * _spmm_sum(s_ref, p_ref)
  mu_ref[...] = t[:, :D_OUT] + b2_ref[...]
  lv_ref[...] = t[:, D_OUT:] + b3_ref[...]


def _row_spec(width, br=_BR):
  return pl.BlockSpec((br, width), lambda i: (i, 0))


def _full_spec(shape):
  return pl.BlockSpec(shape, lambda i: (0,) * len(shape))


def _s_spec(br=_BR):
  return pl.BlockSpec((NC, br, NQ * DQ), lambda i: (0, i, 0))


_p_out = jax.ShapeDtypeStruct((NP, D_HID), jnp.float32)


def _tc1(x_pad, W1, degT):
  return pl.pallas_call(
      _tc1_body,
      grid=(_GRID,),
      in_specs=[_row_spec(128), _full_spec((128, D_HID)), _row_spec(2)],
      out_specs=_row_spec(D_HID),
      out_shape=_p_out,
  )(x_pad, W1, degT)


def _tc2(s1, p, degT, b1, Wc):
  return pl.pallas_call(
      _tc2_body,
      grid=(_GRID,),
      in_specs=[_s_spec(), _row_spec(128), _row_spec(2),
                _full_spec((1, 128)), _full_spec((128, 128))],
      out_specs=_row_spec(D_HID),
      out_shape=_p_out,
  )(s1, p, degT, b1, Wc)


def _tc3(s2, q, degT, b2, b3):
  return pl.pallas_call(
      _tc3_body,
      grid=(_GRID3,),
      in_specs=[_s_spec(_BR3), _row_spec(128, _BR3), _row_spec(2, _BR3),
                _full_spec((1, D_OUT)), _full_spec((1, D_OUT))],
      out_specs=[_row_spec(D_OUT, _BR3), _row_spec(D_OUT, _BR3)],
      out_shape=[jax.ShapeDtypeStruct((N_NODES, D_OUT), jnp.float32)] * 2,
  )(s2, q, degT, b2, b3)


def kernel(x, edge_index, W1, b1, W2, b2, W3, b3):
  ei = edge_index.astype(jnp.int32)
  x_pad = jnp.pad(x, ((0, NP - N_NODES), (0, 0)))
  zero1 = jnp.zeros((NP,), jnp.float32)
  b1r = b1.reshape(1, D_HID)
  b2r = b2.reshape(1, D_OUT)
  b3r = b3.reshape(1, D_OUT)
  Wc = jnp.concatenate([W2, W3], axis=1)

  deg2 = _deg_kernel(ei, zero1)
  degT = deg2.T

  p4 = _tc1(x_pad, W1, degT)
  s1 = _spmm_kernel(ei, p4)
  q4 = _tc2(s1, p4, degT, b1r, Wc)
  s2 = _spmm_kernel(ei, q4)
  return _tc3(s2, q4, degT, b2r, b3r)


# speedup vs baseline: 35.8420x; 1.0213x over previous
import functools

import jax
import jax.numpy as jnp
from jax import lax
from jax.experimental import pallas as pl
from jax.experimental.pallas import tpu as pltpu
from jax.experimental.pallas import tpu_sc as plsc

N_NODES = 10000
N_EDGES = 320000
D_IN = 128
D_HID = 128
D_OUT = 64
DQ = 32
NQ = 4

NC = 2
NS = 16
NW = NC * NS
NP = 10240
RPT = NP // NS
K = 128
NCHT = N_EDGES // K
NCH_LO = NCHT // NW
NREM = NCHT - NCH_LO * NW
NCH_HI = NCH_LO + 1
NCH_UP = NCH_LO + 2

_MESH = plsc.VectorSubcoreMesh(core_axis_name="c", subcore_axis_name="s")
_SC_PARAMS = pltpu.CompilerParams(use_tc_tiling_on_sc=False)


def _chunks(c, s):
  wid = s * NC + c
  base = wid * NCH_LO + jnp.minimum(wid, NREM)
  nch = jnp.where(wid < NREM, NCH_HI, NCH_LO)
  start = base * K
  start_dma = jnp.minimum(start, N_EDGES - NCH_UP * K)
  return start_dma, start - start_dma, nch


def _deg_body(edge_hbm, zero_hbm, out_hbm, idx_d, ones_v, degacc, isem):
  c = lax.axis_index("c")
  s = lax.axis_index("s")
  start_dma, off, nch = _chunks(c, s)
  cp = pltpu.async_copy(edge_hbm.at[1].at[pl.ds(start_dma, NCH_UP * K)], idx_d, isem)
  for i in range(K // 16):
    ones_v[pl.ds(i * 16, 16)] = jnp.full((16,), 1.0, jnp.float32)
  pltpu.sync_copy(zero_hbm.at[pl.ds(s * RPT, RPT)], degacc.at[pl.ds(s * RPT, RPT)])
  cp.wait()
  plsc.subcore_barrier()

  @pl.loop(0, NCH_UP)
  def _(j):
    @pl.when(j < nch)
    def _():
      pltpu.sync_copy(ones_v, degacc.at[idx_d.at[pl.ds(off + j * K, K)]], add=True)

  plsc.subcore_barrier()
  pltpu.sync_copy(degacc.at[pl.ds(s * RPT, RPT)], out_hbm.at[c].at[pl.ds(s * RPT, RPT)])


@functools.partial(
    pl.kernel,
    out_type=jax.ShapeDtypeStruct((NC, NP), jnp.float32),
    mesh=_MESH,
    scratch_types=[
        pltpu.VMEM((NCH_UP * K,), jnp.int32),
        pltpu.VMEM((K,), jnp.float32),
        pltpu.VMEM_SHARED((NP,), jnp.float32),
        pltpu.SemaphoreType.DMA,
    ],
    compiler_params=_SC_PARAMS,
)
def _deg_kernel(edge_hbm, zero_hbm, out_hbm, idx_d, ones_v, degacc, isem):
  _deg_body(edge_hbm, zero_hbm, out_hbm, idx_d, ones_v, degacc, isem)


_ND = 6
_NG = 3


def _spmm_body(edge_hbm, p4_hbm, out_hbm, refs):
  idx_s, idx_d = refs[0], refs[1]
  rows = refs[2:2 + _ND]
  zbuf, pq0, pq1, acc = refs[2 + _ND:6 + _ND]
  isem0, isem1 = refs[6 + _ND], refs[7 + _ND]
  gsem = refs[8 + _ND:8 + 2 * _ND]
  csem = refs[8 + 2 * _ND:8 + 3 * _ND]
  ssem0, ssem1 = refs[8 + 3 * _ND], refs[9 + 3 * _ND]
  c = lax.axis_index("c")
  s = lax.axis_index("s")
  start_dma, off, nch = _chunks(c, s)
  cps = pltpu.async_copy(edge_hbm.at[0].at[pl.ds(start_dma, NCH_UP * K)], idx_s, isem0)
  cpd = pltpu.async_copy(edge_hbm.at[1].at[pl.ds(start_dma, NCH_UP * K)], idx_d, isem1)

  @pl.loop(0, RPT)
  def _(r):
    for cc in range(DQ // 16):
      zbuf[r, pl.ds(cc * 16, 16)] = jnp.zeros((16,), jnp.float32)

  pqs = (pq0, pq1)
  ssem = (ssem0, ssem1)
  rslice = pl.ds(s * RPT, RPT)

  def stage(q, sync):
    cp = pltpu.async_copy(p4_hbm.at[rslice, pl.ds(q * DQ, DQ)],
                          pqs[q % 2].at[rslice], ssem[q % 2])
    if sync:
      cp.wait()

  stage(0, True)
  stage(1, False)
  pltpu.sync_copy(zbuf, acc.at[rslice])
  cps.wait()
  cpd.wait()
  plsc.subcore_barrier()

  def wait_gather(b):
    pltpu.make_async_copy(pqs[0].at[idx_s.at[pl.ds(0, K)]], rows[b],
                          gsem[b]).wait()

  def wait_scatter(b):
    pltpu.make_async_copy(rows[b], acc.at[idx_d.at[pl.ds(0, K)]],
                          csem[b]).wait()

  for q in range(NQ):
    pq = pqs[q % 2]
    for b in range(_NG):
      pltpu.async_copy(pq.at[idx_s.at[pl.ds(off + b * K, K)]], rows[b], gsem[b])

    @pl.loop(0, NCH_UP, step=_ND)
    def _(jj):
      for b in range(_ND):
        j = jj + b

        @pl.when(j < nch)
        def _():
          wait_gather(b)
          pltpu.async_copy(rows[b], acc.at[idx_d.at[pl.ds(off + j * K, K)]],
                           csem[b], add=True)

        @pl.when(j + _NG < nch)
        def _():
          @pl.when(j >= _ND - _NG)
          def _():
            wait_scatter((b + _NG) % _ND)

          pltpu.async_copy(pq.at[idx_s.at[pl.ds(off + (j + _NG) * K, K)]],
                           rows[(b + _NG) % _ND], gsem[(b + _NG) % _ND])

    for b in range(_ND):
      wait_scatter(b)

    plsc.subcore_barrier()
    pltpu.sync_copy(acc.at[rslice], out_hbm.at[c].at[rslice, pl.ds(q * DQ, DQ)])
    if q + 1 < NQ:
      pltpu.sync_copy(zbuf, acc.at[rslice])
      if q + 2 < NQ:
        stage(q + 2, False)
      pltpu.make_async_copy(p4_hbm.at[rslice, pl.ds((q + 1) * DQ, DQ)],
                            pqs[(q + 1) % 2].at[rslice], ssem[(q + 1) % 2]).wait()
      plsc.subcore_barrier()


@functools.partial(
    pl.kernel,
    out_type=jax.ShapeDtypeStruct((NC, NP, NQ * DQ), jnp.float32),
    mesh=_MESH,
    scratch_types=[
        pltpu.VMEM((NCH_UP * K,), jnp.int32),
        pltpu.VMEM((NCH_UP * K,), jnp.int32),
    ] + [pltpu.VMEM((K, DQ), jnp.float32)] * _ND + [
        pltpu.VMEM((RPT, DQ), jnp.float32),
        pltpu.VMEM_SHARED((NP, DQ), jnp.float32),
        pltpu.VMEM_SHARED((NP, DQ), jnp.float32),
        pltpu.VMEM_SHARED((NP, DQ), jnp.float32),
    ] + [pltpu.SemaphoreType.DMA] * (4 + 2 * _ND),
    compiler_params=_SC_PARAMS,
)
def _spmm_kernel(edge_hbm, p4_hbm, out_hbm, *refs):
  _spmm_body(edge_hbm, p4_hbm, out_hbm, refs)


_BR = 1280
_GRID = NP // _BR
_BR3 = 1000
_GRID3 = N_NODES // _BR3


def _rsqrt_deg(d_ref):
  deg = d_ref[:, 0:1] + d_ref[:, 1:2] + 1.0
  return lax.rsqrt(deg)


def _tc1a_body(x_ref, w_ref, o_ref):
  o_ref[...] = jnp.dot(x_ref[...], w_ref[...],
                       preferred_element_type=jnp.float32)


def _tc1b_body(u_ref, d_ref, o_ref):
  o_ref[...] = _rsqrt_deg(d_ref) * u_ref[...]


def _spmm_sum(s_ref, p_ref):
  return s_ref[0] + s_ref[1] + p_ref[...]


def _tc2_body(s_ref, p_ref, d_ref, b_ref, w_ref, o_ref):
  q = _rsqrt_deg(d_ref)
  h = q * _spmm_sum(s_ref, p_ref) + b_ref[...]
  h = jnp.maximum(h, 0.0)
  o_ref[...] = q * jnp.dot(h, w_ref[...], preferred_element_type=jnp.float32)


def _tc3_body(s_ref, p_ref, d_ref, b2_ref, b3_ref, mu_ref, lv_ref):
  q = _rsqrt_deg(d_ref)
  t = q * _spmm_sum(s_ref, p_ref)
  mu_ref[...] = t[:, :D_OUT] + b2_ref[...]
  lv_ref[...] = t[:, D_OUT:] + b3_ref[...]


def _row_spec(width, br=_BR):
  return pl.BlockSpec((br, width), lambda i: (i, 0))


def _full_spec(shape):
  return pl.BlockSpec(shape, lambda i: (0,) * len(shape))


def _s_spec(br=_BR):
  return pl.BlockSpec((NC, br, NQ * DQ), lambda i: (0, i, 0))


_p_out = jax.ShapeDtypeStruct((NP, D_HID), jnp.float32)


def _tc1a(x_pad, W1):
  return pl.pallas_call(
      _tc1a_body,
      grid=(_GRID,),
      in_specs=[_row_spec(128), _full_spec((128, D_HID))],
      out_specs=_row_spec(D_HID),
      out_shape=_p_out,
  )(x_pad, W1)


def _tc1b(u, degT):
  return pl.pallas_call(
      _tc1b_body,
      grid=(_GRID,),
      in_specs=[_row_spec(128), _row_spec(2)],
      out_specs=_row_spec(D_HID),
      out_shape=_p_out,
  )(u, degT)


def _tc2(s1, p, degT, b1, Wc):
  return pl.pallas_call(
      _tc2_body,
      grid=(_GRID,),
      in_specs=[_s_spec(), _row_spec(128), _row_spec(2),
                _full_spec((1, 128)), _full_spec((128, 128))],
      out_specs=_row_spec(D_HID),
      out_shape=_p_out,
  )(s1, p, degT, b1, Wc)


def _tc3(s2, q, degT, b2, b3):
  return pl.pallas_call(
      _tc3_body,
      grid=(_GRID3,),
      in_specs=[_s_spec(_BR3), _row_spec(128, _BR3), _row_spec(2, _BR3),
                _full_spec((1, D_OUT)), _full_spec((1, D_OUT))],
      out_specs=[_row_spec(D_OUT, _BR3), _row_spec(D_OUT, _BR3)],
      out_shape=[jax.ShapeDtypeStruct((N_NODES, D_OUT), jnp.float32)] * 2,
  )(s2, q, degT, b2, b3)


def kernel(x, edge_index, W1, b1, W2, b2, W3, b3):
  ei = edge_index.astype(jnp.int32)
  x_pad = jnp.pad(x, ((0, NP - N_NODES), (0, 0)))
  zero1 = jnp.zeros((NP,), jnp.float32)
  b1r = b1.reshape(1, D_HID)
  b2r = b2.reshape(1, D_OUT)
  b3r = b3.reshape(1, D_OUT)
  Wc = jnp.concatenate([W2, W3], axis=1)

  deg2 = _deg_kernel(ei, zero1)
  degT = deg2.T
  u = _tc1a(x_pad, W1)
  p4 = _tc1b(u, degT)
  s1 = _spmm_kernel(ei, p4)
  q4 = _tc2(s1, p4, degT, b1r, Wc)
  s2 = _spmm_kernel(ei, q4)
  return _tc3(s2, q4, degT, b2r, b3r)


# speedup vs baseline: 35.8527x; 1.0003x over previous
import functools

import jax
import jax.numpy as jnp
from jax import lax
from jax.experimental import pallas as pl
from jax.experimental.pallas import tpu as pltpu
from jax.experimental.pallas import tpu_sc as plsc

N_NODES = 10000
N_EDGES = 320000
D_IN = 128
D_HID = 128
D_OUT = 64
DQ = 32
NQ = 4

NC = 2
NS = 16
NW = NC * NS
NP = 10240
RPT = NP // NS
K = 128
NCHT = N_EDGES // K
NCH_LO = NCHT // NW
NREM = NCHT - NCH_LO * NW
NCH_HI = NCH_LO + 1
NCH_UP = NCH_LO + 2

_MESH = plsc.VectorSubcoreMesh(core_axis_name="c", subcore_axis_name="s")
_SC_PARAMS = pltpu.CompilerParams(use_tc_tiling_on_sc=False)


def _chunks(c, s):
  wid = s * NC + c
  base = wid * NCH_LO + jnp.minimum(wid, NREM)
  nch = jnp.where(wid < NREM, NCH_HI, NCH_LO)
  start = base * K
  start_dma = jnp.minimum(start, N_EDGES - NCH_UP * K)
  return start_dma, start - start_dma, nch


def _deg_body(edge_hbm, zero_hbm, out_hbm, idx_d, ones_v, degacc, isem):
  c = lax.axis_index("c")
  s = lax.axis_index("s")
  start_dma, off, nch = _chunks(c, s)
  cp = pltpu.async_copy(edge_hbm.at[1].at[pl.ds(start_dma, NCH_UP * K)], idx_d, isem)
  for i in range(K // 16):
    ones_v[pl.ds(i * 16, 16)] = jnp.full((16,), 1.0, jnp.float32)
  pltpu.sync_copy(zero_hbm.at[pl.ds(s * RPT, RPT)], degacc.at[pl.ds(s * RPT, RPT)])
  cp.wait()
  plsc.subcore_barrier()

  @pl.loop(0, NCH_UP)
  def _(j):
    @pl.when(j < nch)
    def _():
      pltpu.sync_copy(ones_v, degacc.at[idx_d.at[pl.ds(off + j * K, K)]], add=True)

  plsc.subcore_barrier()
  pltpu.sync_copy(degacc.at[pl.ds(s * RPT, RPT)], out_hbm.at[c].at[pl.ds(s * RPT, RPT)])


@functools.partial(
    pl.kernel,
    out_type=jax.ShapeDtypeStruct((NC, NP), jnp.float32),
    mesh=_MESH,
    scratch_types=[
        pltpu.VMEM((NCH_UP * K,), jnp.int32),
        pltpu.VMEM((K,), jnp.float32),
        pltpu.VMEM_SHARED((NP,), jnp.float32),
        pltpu.SemaphoreType.DMA,
    ],
    compiler_params=_SC_PARAMS,
)
def _deg_kernel(edge_hbm, zero_hbm, out_hbm, idx_d, ones_v, degacc, isem):
  _deg_body(edge_hbm, zero_hbm, out_hbm, idx_d, ones_v, degacc, isem)


_ND = 6
_NG = 4


def _spmm_body(edge_hbm, p4_hbm, out_hbm, refs):
  idx_s, idx_d = refs[0], refs[1]
  rows = refs[2:2 + _ND]
  zbuf, pq0, pq1, acc = refs[2 + _ND:6 + _ND]
  isem0, isem1 = refs[6 + _ND], refs[7 + _ND]
  gsem = refs[8 + _ND:8 + 2 * _ND]
  csem = refs[8 + 2 * _ND:8 + 3 * _ND]
  ssem0, ssem1 = refs[8 + 3 * _ND], refs[9 + 3 * _ND]
  c = lax.axis_index("c")
  s = lax.axis_index("s")
  start_dma, off, nch = _chunks(c, s)
  cps = pltpu.async_copy(edge_hbm.at[0].at[pl.ds(start_dma, NCH_UP * K)], idx_s, isem0)
  cpd = pltpu.async_copy(edge_hbm.at[1].at[pl.ds(start_dma, NCH_UP * K)], idx_d, isem1)

  @pl.loop(0, RPT)
  def _(r):
    for cc in range(DQ // 16):
      zbuf[r, pl.ds(cc * 16, 16)] = jnp.zeros((16,), jnp.float32)

  pqs = (pq0, pq1)
  ssem = (ssem0, ssem1)
  rslice = pl.ds(s * RPT, RPT)

  def stage(q, sync):
    cp = pltpu.async_copy(p4_hbm.at[rslice, pl.ds(q * DQ, DQ)],
                          pqs[q % 2].at[rslice], ssem[q % 2])
    if sync:
      cp.wait()

  stage(0, True)
  stage(1, False)
  pltpu.sync_copy(zbuf, acc.at[rslice])
  cps.wait()
  cpd.wait()
  plsc.subcore_barrier()

  def wait_gather(b):
    pltpu.make_async_copy(pqs[0].at[idx_s.at[pl.ds(0, K)]], rows[b],
                          gsem[b]).wait()

  def wait_scatter(b):
    pltpu.make_async_copy(rows[b], acc.at[idx_d.at[pl.ds(0, K)]],
                          csem[b]).wait()

  for q in range(NQ):
    pq = pqs[q % 2]
    for b in range(_NG):
      pltpu.async_copy(pq.at[idx_s.at[pl.ds(off + b * K, K)]], rows[b], gsem[b])

    @pl.loop(0, NCH_UP, step=_ND)
    def _(jj):
      for b in range(_ND):
        j = jj + b

        @pl.when(j < nch)
        def _():
          wait_gather(b)
          pltpu.async_copy(rows[b], acc.at[idx_d.at[pl.ds(off + j * K, K)]],
                           csem[b], add=True)

        @pl.when(j + _NG < nch)
        def _():
          @pl.when(j >= _ND - _NG)
          def _():
            wait_scatter((b + _NG) % _ND)

          pltpu.async_copy(pq.at[idx_s.at[pl.ds(off + (j + _NG) * K, K)]],
                           rows[(b + _NG) % _ND], gsem[(b + _NG) % _ND])

    for b in range(_ND):
      wait_scatter(b)

    plsc.subcore_barrier()
    pltpu.sync_copy(acc.at[rslice], out_hbm.at[c].at[rslice, pl.ds(q * DQ, DQ)])
    if q + 1 < NQ:
      pltpu.sync_copy(zbuf, acc.at[rslice])
      if q + 2 < NQ:
        stage(q + 2, False)
      pltpu.make_async_copy(p4_hbm.at[rslice, pl.ds((q + 1) * DQ, DQ)],
                            pqs[(q + 1) % 2].at[rslice], ssem[(q + 1) % 2]).wait()
      plsc.subcore_barrier()


@functools.partial(
    pl.kernel,
    out_type=jax.ShapeDtypeStruct((NC, NP, NQ * DQ), jnp.float32),
    mesh=_MESH,
    scratch_types=[
        pltpu.VMEM((NCH_UP * K,), jnp.int32),
        pltpu.VMEM((NCH_UP * K,), jnp.int32),
    ] + [pltpu.VMEM((K, DQ), jnp.float32)] * _ND + [
        pltpu.VMEM((RPT, DQ), jnp.float32),
        pltpu.VMEM_SHARED((NP, DQ), jnp.float32),
        pltpu.VMEM_SHARED((NP, DQ), jnp.float32),
        pltpu.VMEM_SHARED((NP, DQ), jnp.float32),
    ] + [pltpu.SemaphoreType.DMA] * (4 + 2 * _ND),
    compiler_params=_SC_PARAMS,
)
def _spmm_kernel(edge_hbm, p4_hbm, out_hbm, *refs):
  _spmm_body(edge_hbm, p4_hbm, out_hbm, refs)


_BR = 1280
_GRID = NP // _BR
_BR3 = 1000
_GRID3 = N_NODES // _BR3


def _rsqrt_deg(d_ref):
  deg = d_ref[:, 0:1] + d_ref[:, 1:2] + 1.0
  return lax.rsqrt(deg)


def _tc1a_body(x_ref, w_ref, o_ref):
  o_ref[...] = jnp.dot(x_ref[...], w_ref[...],
                       preferred_element_type=jnp.float32)


def _tc1b_body(u_ref, d_ref, o_ref):
  o_ref[...] = _rsqrt_deg(d_ref) * u_ref[...]


def _spmm_sum(s_ref, p_ref):
  return s_ref[0] + s_ref[1] + p_ref[...]


def _tc2_body(s_ref, p_ref, d_ref, b_ref, w_ref, o_ref):
  q = _rsqrt_deg(d_ref)
  h = q * _spmm_sum(s_ref, p_ref) + b_ref[...]
  h = jnp.maximum(h, 0.0)
  o_ref[...] = q * jnp.dot(h, w_ref[...], preferred_element_type=jnp.float32)


def _tc3_body(s_ref, p_ref, d_ref, b2_ref, b3_ref, mu_ref, lv_ref):
  q = _rsqrt_deg(d_ref)
  t = q * _spmm_sum(s_ref, p_ref)
  mu_ref[...] = t[:, :D_OUT] + b2_ref[...]
  lv_ref[...] = t[:, D_OUT:] + b3_ref[...]


def _row_spec(width, br=_BR):
  return pl.BlockSpec((br, width), lambda i: (i, 0))


def _full_spec(shape):
  return pl.BlockSpec(shape, lambda i: (0,) * len(shape))


def _s_spec(br=_BR):
  return pl.BlockSpec((NC, br, NQ * DQ), lambda i: (0, i, 0))


_p_out = jax.ShapeDtypeStruct((NP, D_HID), jnp.float32)


def _tc1a(x_pad, W1):
  return pl.pallas_call(
      _tc1a_body,
      grid=(_GRID,),
      in_specs=[_row_spec(128), _full_spec((128, D_HID))],
      out_specs=_row_spec(D_HID),
      out_shape=_p_out,
  )(x_pad, W1)


def _tc1b(u, degT):
  return pl.pallas_call(
      _tc1b_body,
      grid=(_GRID,),
      in_specs=[_row_spec(128), _row_spec(2)],
      out_specs=_row_spec(D_HID),
      out_shape=_p_out,
  )(u, degT)


def _tc2(s1, p, degT, b1, Wc):
  return pl.pallas_call(
      _tc2_body,
      grid=(_GRID,),
      in_specs=[_s_spec(), _row_spec(128), _row_spec(2),
                _full_spec((1, 128)), _full_spec((128, 128))],
      out_specs=_row_spec(D_HID),
      out_shape=_p_out,
  )(s1, p, degT, b1, Wc)


def _tc3(s2, q, degT, b2, b3):
  return pl.pallas_call(
      _tc3_body,
      grid=(_GRID3,),
      in_specs=[_s_spec(_BR3), _row_spec(128, _BR3), _row_spec(2, _BR3),
                _full_spec((1, D_OUT)), _full_spec((1, D_OUT))],
      out_specs=[_row_spec(D_OUT, _BR3), _row_spec(D_OUT, _BR3)],
      out_shape=[jax.ShapeDtypeStruct((N_NODES, D_OUT), jnp.float32)] * 2,
  )(s2, q, degT, b2, b3)


def kernel(x, edge_index, W1, b1, W2, b2, W3, b3):
  ei = edge_index.astype(jnp.int32)
  x_pad = jnp.pad(x, ((0, NP - N_NODES), (0, 0)))
  zero1 = jnp.zeros((NP,), jnp.float32)
  b1r = b1.reshape(1, D_HID)
  b2r = b2.reshape(1, D_OUT)
  b3r = b3.reshape(1, D_OUT)
  Wc = jnp.concatenate([W2, W3], axis=1)

  deg2 = _deg_kernel(ei, zero1)
  degT = deg2.T
  u = _tc1a(x_pad, W1)
  p4 = _tc1b(u, degT)
  s1 = _spmm_kernel(ei, p4)
  q4 = _tc2(s1, p4, degT, b1r, Wc)
  s2 = _spmm_kernel(ei, q4)
  return _tc3(s2, q4, degT, b2r, b3r)
